# baseline (speedup 1.0000x reference)
import functools
import math

import jax
import jax.numpy as jnp
from jax import lax
from jax.experimental import pallas as pl
from jax.experimental.pallas import tpu as pltpu
from jax.experimental.pallas import tpu_sc as plsc

N = 10000
E = 320000
NE = 10000
F = 128
H = 128
C = 128
DEPTH = 4
ALPHA = 0.1

NC = 2
NS = 16
NW = NC * NS
EPW = E // NW
B = 128
PH = 6
PCH = 13
FCH = PH * PCH
TB = EPW - FCH * B
NP = 10240
STRIPE = NP // NS

_mesh = plsc.VectorSubcoreMesh(core_axis_name="c", subcore_axis_name="s")


@functools.partial(
    pl.kernel,
    out_type=jax.ShapeDtypeStruct((NC, NP, H), jnp.float32),
    mesh=_mesh,
    scratch_types=[
        pltpu.VMEM((PCH, B), jnp.int32),
        pltpu.VMEM((PCH, B), jnp.int32),
        pltpu.VMEM((B,), jnp.int32),
        pltpu.VMEM((B,), jnp.int32),
        pltpu.VMEM((TB,), jnp.int32),
        pltpu.VMEM((TB,), jnp.int32),
        pltpu.VMEM((B, H), jnp.float32),
        pltpu.VMEM((B, H), jnp.float32),
        pltpu.VMEM_SHARED((NP, H), jnp.float32),
        pltpu.SemaphoreType.DMA,
        pltpu.SemaphoreType.DMA,
        pltpu.SemaphoreType.DMA,
        pltpu.SemaphoreType.DMA,
    ],
)
def _sc_agg(src, idxg, idxs, idxgt, idxst, zrows, out,
            idxg_v, idxs_v, s0, s1, tg, ts, r0, r1, acc, g0, g1, a0, a1):
    cid = lax.axis_index("c")
    sid = lax.axis_index("s")
    wid = sid * NC + cid
    pltpu.sync_copy(zrows, acc.at[pl.ds(sid * STRIPE, STRIPE)])
    plsc.subcore_barrier()

    def stage(c, dst):
        for k in range(B // 16):
            dst[pl.ds(16 * k, 16)] = idxs_v[c, pl.ds(16 * k, 16)]

    def gwait(c, r, g):
        pltpu.make_async_copy(src.at[idxg_v.at[c]], r, g).wait()

    def swait(r, s, a):
        pltpu.make_async_copy(r, acc.at[s], a).wait()

    def phase(p, carry):
        pltpu.sync_copy(idxg.at[wid, p], idxg_v)
        pltpu.sync_copy(idxs.at[wid, p], idxs_v)
        stage(0, s0)
        pltpu.async_copy(src.at[idxg_v.at[0]], r0, g0)
        stage(1, s1)
        pltpu.async_copy(src.at[idxg_v.at[1]], r1, g1)

        def pair(g, carry2):
            c0 = 2 * g
            gwait(c0, r0, g0)
            pltpu.async_copy(r0, acc.at[s0], a0, add=True)
            gwait(c0 + 1, r1, g1)
            pltpu.async_copy(r1, acc.at[s1], a1, add=True)
            swait(r0, s0, a0)
            stage(c0 + 2, s0)
            pltpu.async_copy(src.at[idxg_v.at[c0 + 2]], r0, g0)
            swait(r1, s1, a1)
            stage(c0 + 3, s1)
            pltpu.async_copy(src.at[idxg_v.at[c0 + 3]], r1, g1)
            return carry2

        lax.fori_loop(0, (PCH - 3) // 2, pair, 0)
        gwait(PCH - 3, r0, g0)
        pltpu.async_copy(r0, acc.at[s0], a0, add=True)
        gwait(PCH - 2, r1, g1)
        pltpu.async_copy(r1, acc.at[s1], a1, add=True)
        swait(r0, s0, a0)
        stage(PCH - 1, s0)
        pltpu.async_copy(src.at[idxg_v.at[PCH - 1]], r0, g0)
        gwait(PCH - 1, r0, g0)
        pltpu.async_copy(r0, acc.at[s0], a0, add=True)
        swait(r0, s0, a0)
        swait(r1, s1, a1)
        return carry

    lax.fori_loop(0, PH, phase, 0)

    pltpu.sync_copy(idxgt.at[wid], tg)
    pltpu.sync_copy(idxst.at[wid], ts)
    pltpu.async_copy(src.at[tg], r0.at[pl.ds(0, TB)], g0)
    pltpu.make_async_copy(src.at[tg], r0.at[pl.ds(0, TB)], g0).wait()
    pltpu.sync_copy(r0.at[pl.ds(0, TB)], acc.at[ts], add=True)

    plsc.subcore_barrier()
    pltpu.sync_copy(acc.at[pl.ds(sid * STRIPE, STRIPE)],
                    out.at[cid, pl.ds(sid * STRIPE, STRIPE)])


@functools.partial(
    pl.kernel,
    out_type=(jax.ShapeDtypeStruct((NC, NP, 16), jnp.float32),
              jax.ShapeDtypeStruct((NC, NP, 16), jnp.float32)),
    mesh=_mesh,
    scratch_types=[
        pltpu.VMEM((PCH, B), jnp.int32),
        pltpu.VMEM((PCH, B), jnp.int32),
        pltpu.VMEM((B,), jnp.int32),
        pltpu.VMEM((B,), jnp.int32),
        pltpu.VMEM((B,), jnp.int32),
        pltpu.VMEM((B,), jnp.int32),
        pltpu.VMEM((TB,), jnp.int32),
        pltpu.VMEM((TB,), jnp.int32),
        pltpu.VMEM((B, 16), jnp.float32),
        pltpu.VMEM_SHARED((NP, 16), jnp.float32),
        pltpu.VMEM_SHARED((NP, 16), jnp.float32),
        pltpu.SemaphoreType.DMA,
        pltpu.SemaphoreType.DMA,
        pltpu.SemaphoreType.DMA,
        pltpu.SemaphoreType.DMA,
    ],
)
def _sc_count(idxe, idxv, idxet, idxvt, z16, ones_hbm, oute, outv,
              idxe_v, idxv_v, se0, se1, sv0, sv1, te, tv, ones_v,
              acce, accv, ae0, av0, ae1, av1):
    cid = lax.axis_index("c")
    sid = lax.axis_index("s")
    wid = sid * NC + cid
    pltpu.sync_copy(ones_hbm, ones_v)
    pltpu.sync_copy(z16, acce.at[pl.ds(sid * STRIPE, STRIPE)])
    pltpu.sync_copy(z16, accv.at[pl.ds(sid * STRIPE, STRIPE)])
    plsc.subcore_barrier()

    def stage(ref, c, dst):
        for k in range(B // 16):
            dst[pl.ds(16 * k, 16)] = ref[c, pl.ds(16 * k, 16)]

    def issue(se, sv, semee, semv):
        pltpu.async_copy(ones_v, acce.at[se], semee, add=True)
        pltpu.async_copy(ones_v, accv.at[sv], semv, add=True)

    def drain(se, sv, semee, semv):
        pltpu.make_async_copy(ones_v, acce.at[se], semee).wait()
        pltpu.make_async_copy(ones_v, accv.at[sv], semv).wait()

    def phase(p, carry):
        pltpu.sync_copy(idxe.at[wid, p], idxe_v)
        pltpu.sync_copy(idxv.at[wid, p], idxv_v)
        stage(idxe_v, 0, se0)
        stage(idxv_v, 0, sv0)
        issue(se0, sv0, ae0, av0)
        stage(idxe_v, 1, se1)
        stage(idxv_v, 1, sv1)
        issue(se1, sv1, ae1, av1)

        def pair(g, carry2):
            c = 2 * g + 2
            drain(se0, sv0, ae0, av0)
            stage(idxe_v, c, se0)
            stage(idxv_v, c, sv0)
            issue(se0, sv0, ae0, av0)
            drain(se1, sv1, ae1, av1)
            stage(idxe_v, c + 1, se1)
            stage(idxv_v, c + 1, sv1)
            issue(se1, sv1, ae1, av1)
            return carry2

        lax.fori_loop(0, (PCH - 3) // 2, pair, 0)
        drain(se0, sv0, ae0, av0)
        stage(idxe_v, PCH - 1, se0)
        stage(idxv_v, PCH - 1, sv0)
        issue(se0, sv0, ae0, av0)
        drain(se0, sv0, ae0, av0)
        drain(se1, sv1, ae1, av1)
        return carry

    lax.fori_loop(0, PH, phase, 0)

    pltpu.sync_copy(idxet.at[wid], te)
    pltpu.sync_copy(idxvt.at[wid], tv)
    pltpu.sync_copy(ones_v.at[pl.ds(0, TB)], acce.at[te], add=True)
    pltpu.sync_copy(ones_v.at[pl.ds(0, TB)], accv.at[tv], add=True)

    plsc.subcore_barrier()
    pltpu.sync_copy(acce.at[pl.ds(sid * STRIPE, STRIPE)],
                    oute.at[cid, pl.ds(sid * STRIPE, STRIPE)])
    pltpu.sync_copy(accv.at[pl.ds(sid * STRIPE, STRIPE)],
                    outv.at[cid, pl.ds(sid * STRIPE, STRIPE)])


BR = 400
GR = N // BR


def _linear_body(relu, x_ref, w_ref, b_ref, o_ref):
    acc = lax.dot_general(x_ref[...], w_ref[...], (((1,), (1,)), ((), ())),
                          preferred_element_type=jnp.float32)
    acc = acc + b_ref[...]
    o_ref[...] = jnp.maximum(acc, 0.0) if relu else acc


def _tc_linear(x, w, b, relu):
    return pl.pallas_call(
        functools.partial(_linear_body, relu),
        grid=(GR,),
        in_specs=[
            pl.BlockSpec((BR, x.shape[1]), lambda i: (i, 0)),
            pl.BlockSpec(w.shape, lambda i: (0, 0)),
            pl.BlockSpec((1, w.shape[0]), lambda i: (0, 0)),
        ],
        out_specs=pl.BlockSpec((BR, w.shape[0]), lambda i: (i, 0)),
        out_shape=jax.ShapeDtypeStruct((x.shape[0], w.shape[0]), jnp.float32),
    )(x, w, b.reshape(1, -1))


def _combine_body(p_ref, c_ref, o_ref):
    s = p_ref[0] + p_ref[1]
    cnt = c_ref[0, :, 0:1] + c_ref[1, :, 0:1]
    o_ref[...] = s * (1.0 / jnp.maximum(cnt, 1.0))


def _tc_combine(partials, cnts):
    return pl.pallas_call(
        _combine_body,
        grid=(GR,),
        in_specs=[
            pl.BlockSpec((NC, BR, H), lambda i: (0, i, 0)),
            pl.BlockSpec((NC, BR, 16), lambda i: (0, i, 0)),
        ],
        out_specs=pl.BlockSpec((BR, H), lambda i: (i, 0)),
        out_shape=jax.ShapeDtypeStruct((NE, H), jnp.float32),
    )(partials, cnts)


def _layer_body(beta, p_ref, c_ref, h0_ref, w_ref, o_ref):
    s = p_ref[0] + p_ref[1]
    cnt = c_ref[0, :, 0:1] + c_ref[1, :, 0:1]
    xv = s * (1.0 / jnp.maximum(cnt, 1.0))
    rn = jnp.sqrt(jnp.sum(xv * xv, axis=1, keepdims=True))
    xn = xv * jnp.where(rn > 0, 1.0 / rn, 0.0)
    xi = (1.0 - ALPHA) * xn + ALPHA * h0_ref[...]
    xw = lax.dot_general(xi, w_ref[...], (((1,), (1,)), ((), ())),
                         preferred_element_type=jnp.float32)
    o_ref[...] = jnp.maximum((1.0 - beta) * xi + beta * xw, 0.0)


def _tc_layer(partials, cnts, h0, w, beta):
    return pl.pallas_call(
        functools.partial(_layer_body, beta),
        grid=(GR,),
        in_specs=[
            pl.BlockSpec((NC, BR, H), lambda i: (0, i, 0)),
            pl.BlockSpec((NC, BR, 16), lambda i: (0, i, 0)),
            pl.BlockSpec((BR, H), lambda i: (i, 0)),
            pl.BlockSpec((H, H), lambda i: (0, 0)),
        ],
        out_specs=pl.BlockSpec((BR, H), lambda i: (i, 0)),
        out_shape=jax.ShapeDtypeStruct((N, H), jnp.float32),
    )(partials, cnts, h0, w)


def kernel(x, edge_index, W0, b0, Ws, Wout, bout):
    vw = edge_index[0].reshape(NW, EPW)
    ew = edge_index[1].reshape(NW, EPW)
    vertex_m = vw[:, :FCH * B].reshape(NW, PH, PCH, B)
    edges_m = ew[:, :FCH * B].reshape(NW, PH, PCH, B)
    vertex_t = vw[:, FCH * B:]
    edges_t = ew[:, FCH * B:]
    zrows = jnp.zeros((STRIPE, H), jnp.float32)
    z16 = jnp.zeros((STRIPE, 16), jnp.float32)
    ones = jnp.ones((B, 16), jnp.float32)

    h = _tc_linear(x, W0, b0, relu=True)
    h0 = h
    ce, cv = _sc_count(edges_m, vertex_m, edges_t, vertex_t, z16, ones)
    for i in range(DEPTH):
        beta = math.log(0.5 / (i + 1) + 1.0)
        sep = _sc_agg(h, vertex_m, edges_m, vertex_t, edges_t, zrows)
        xe = _tc_combine(sep, ce)
        svp = _sc_agg(xe, edges_m, vertex_m, edges_t, vertex_t, zrows)
        h = _tc_layer(svp, cv, h0, Ws[i], beta)
    return _tc_linear(h, Wout, bout, relu=False)


# speedup vs baseline: 1.2082x; 1.2082x over previous
import functools
import math

import jax
import jax.numpy as jnp
from jax import lax
from jax.experimental import pallas as pl
from jax.experimental.pallas import tpu as pltpu
from jax.experimental.pallas import tpu_sc as plsc

N = 10000
E = 320000
NE = 10000
F = 128
H = 128
C = 128
DEPTH = 4
ALPHA = 0.1

NC = 2
NS = 16
NW = NC * NS
EPW = E // NW
B = 128
PH = 6
PCH = 13
FCH = PH * PCH
TB = EPW - FCH * B
NP = 10240
STRIPE = NP // NS

_mesh = plsc.VectorSubcoreMesh(core_axis_name="c", subcore_axis_name="s")


@functools.partial(
    pl.kernel,
    out_type=jax.ShapeDtypeStruct((NC, NP, H), jnp.float32),
    mesh=_mesh,
    scratch_types=[
        pltpu.VMEM((PCH, B), jnp.int32),
        pltpu.VMEM((PCH, B), jnp.int32),
        pltpu.VMEM((B,), jnp.int32),
        pltpu.VMEM((B,), jnp.int32),
        pltpu.VMEM((TB,), jnp.int32),
        pltpu.VMEM((TB,), jnp.int32),
        pltpu.VMEM((B, H), jnp.float32),
        pltpu.VMEM((B, H), jnp.float32),
        pltpu.VMEM_SHARED((NP, H), jnp.float32),
        pltpu.SemaphoreType.DMA,
        pltpu.SemaphoreType.DMA,
    ],
)
def _sc_agg(src, idxg, idxs, idxgt, idxst, zrows, out,
            idxg_v, idxs_v, s0, s1, tg, ts, r0, r1, acc, g0, g1):
    cid = lax.axis_index("c")
    sid = lax.axis_index("s")
    wid = sid * NC + cid
    pltpu.sync_copy(zrows, acc.at[pl.ds(sid * STRIPE, STRIPE)])
    plsc.subcore_barrier()

    def stage(c, dst):
        for k in range(B // 16):
            dst[pl.ds(16 * k, 16)] = idxs_v[c, pl.ds(16 * k, 16)]

    def phase(p, carry):
        pltpu.sync_copy(idxg.at[wid, p], idxg_v)
        pltpu.sync_copy(idxs.at[wid, p], idxs_v)
        stage(0, s0)
        pltpu.async_copy(src.at[idxg_v.at[0]], r0, g0)

        def pair(g, carry2):
            c0 = 2 * g
            stage(c0 + 1, s1)
            pltpu.async_copy(src.at[idxg_v.at[c0 + 1]], r1, g1)
            pltpu.make_async_copy(src.at[idxg_v.at[c0]], r0, g0).wait()
            pltpu.sync_copy(r0, acc.at[s0], add=True)
            stage(c0 + 2, s0)
            pltpu.async_copy(src.at[idxg_v.at[c0 + 2]], r0, g0)
            pltpu.make_async_copy(src.at[idxg_v.at[c0 + 1]], r1, g1).wait()
            pltpu.sync_copy(r1, acc.at[s1], add=True)
            return carry2

        lax.fori_loop(0, (PCH - 1) // 2, pair, 0)
        pltpu.make_async_copy(src.at[idxg_v.at[PCH - 1]], r0, g0).wait()
        pltpu.sync_copy(r0, acc.at[s0], add=True)
        return carry

    lax.fori_loop(0, PH, phase, 0)

    pltpu.sync_copy(idxgt.at[wid], tg)
    pltpu.sync_copy(idxst.at[wid], ts)
    pltpu.async_copy(src.at[tg], r0.at[pl.ds(0, TB)], g0)
    pltpu.make_async_copy(src.at[tg], r0.at[pl.ds(0, TB)], g0).wait()
    pltpu.sync_copy(r0.at[pl.ds(0, TB)], acc.at[ts], add=True)

    plsc.subcore_barrier()
    pltpu.sync_copy(acc.at[pl.ds(sid * STRIPE, STRIPE)],
                    out.at[cid, pl.ds(sid * STRIPE, STRIPE)])


@functools.partial(
    pl.kernel,
    out_type=(jax.ShapeDtypeStruct((NC, NP, 16), jnp.float32),
              jax.ShapeDtypeStruct((NC, NP, 16), jnp.float32)),
    mesh=_mesh,
    scratch_types=[
        pltpu.VMEM((PCH, B), jnp.int32),
        pltpu.VMEM((PCH, B), jnp.int32),
        pltpu.VMEM((B,), jnp.int32),
        pltpu.VMEM((B,), jnp.int32),
        pltpu.VMEM((B,), jnp.int32),
        pltpu.VMEM((B,), jnp.int32),
        pltpu.VMEM((TB,), jnp.int32),
        pltpu.VMEM((TB,), jnp.int32),
        pltpu.VMEM((B, 16), jnp.float32),
        pltpu.VMEM_SHARED((NP, 16), jnp.float32),
        pltpu.VMEM_SHARED((NP, 16), jnp.float32),
        pltpu.SemaphoreType.DMA,
        pltpu.SemaphoreType.DMA,
        pltpu.SemaphoreType.DMA,
        pltpu.SemaphoreType.DMA,
    ],
)
def _sc_count(idxe, idxv, idxet, idxvt, z16, ones_hbm, oute, outv,
              idxe_v, idxv_v, se0, se1, sv0, sv1, te, tv, ones_v,
              acce, accv, ae0, av0, ae1, av1):
    cid = lax.axis_index("c")
    sid = lax.axis_index("s")
    wid = sid * NC + cid
    pltpu.sync_copy(ones_hbm, ones_v)
    pltpu.sync_copy(z16, acce.at[pl.ds(sid * STRIPE, STRIPE)])
    pltpu.sync_copy(z16, accv.at[pl.ds(sid * STRIPE, STRIPE)])
    plsc.subcore_barrier()

    def stage(ref, c, dst):
        for k in range(B // 16):
            dst[pl.ds(16 * k, 16)] = ref[c, pl.ds(16 * k, 16)]

    def issue(se, sv, semee, semv):
        pltpu.async_copy(ones_v, acce.at[se], semee, add=True)
        pltpu.async_copy(ones_v, accv.at[sv], semv, add=True)

    def drain(se, sv, semee, semv):
        pltpu.make_async_copy(ones_v, acce.at[se], semee).wait()
        pltpu.make_async_copy(ones_v, accv.at[sv], semv).wait()

    def phase(p, carry):
        pltpu.sync_copy(idxe.at[wid, p], idxe_v)
        pltpu.sync_copy(idxv.at[wid, p], idxv_v)
        stage(idxe_v, 0, se0)
        stage(idxv_v, 0, sv0)
        issue(se0, sv0, ae0, av0)
        stage(idxe_v, 1, se1)
        stage(idxv_v, 1, sv1)
        issue(se1, sv1, ae1, av1)

        def pair(g, carry2):
            c = 2 * g + 2
            drain(se0, sv0, ae0, av0)
            stage(idxe_v, c, se0)
            stage(idxv_v, c, sv0)
            issue(se0, sv0, ae0, av0)
            drain(se1, sv1, ae1, av1)
            stage(idxe_v, c + 1, se1)
            stage(idxv_v, c + 1, sv1)
            issue(se1, sv1, ae1, av1)
            return carry2

        lax.fori_loop(0, (PCH - 3) // 2, pair, 0)
        drain(se0, sv0, ae0, av0)
        stage(idxe_v, PCH - 1, se0)
        stage(idxv_v, PCH - 1, sv0)
        issue(se0, sv0, ae0, av0)
        drain(se0, sv0, ae0, av0)
        drain(se1, sv1, ae1, av1)
        return carry

    lax.fori_loop(0, PH, phase, 0)

    pltpu.sync_copy(idxet.at[wid], te)
    pltpu.sync_copy(idxvt.at[wid], tv)
    pltpu.sync_copy(ones_v.at[pl.ds(0, TB)], acce.at[te], add=True)
    pltpu.sync_copy(ones_v.at[pl.ds(0, TB)], accv.at[tv], add=True)

    plsc.subcore_barrier()
    pltpu.sync_copy(acce.at[pl.ds(sid * STRIPE, STRIPE)],
                    oute.at[cid, pl.ds(sid * STRIPE, STRIPE)])
    pltpu.sync_copy(accv.at[pl.ds(sid * STRIPE, STRIPE)],
                    outv.at[cid, pl.ds(sid * STRIPE, STRIPE)])


BR = 400
GR = N // BR


def _linear_body(relu, x_ref, w_ref, b_ref, o_ref):
    acc = lax.dot_general(x_ref[...], w_ref[...], (((1,), (1,)), ((), ())),
                          preferred_element_type=jnp.float32)
    acc = acc + b_ref[...]
    o_ref[...] = jnp.maximum(acc, 0.0) if relu else acc


def _tc_linear(x, w, b, relu):
    return pl.pallas_call(
        functools.partial(_linear_body, relu),
        grid=(GR,),
        in_specs=[
            pl.BlockSpec((BR, x.shape[1]), lambda i: (i, 0)),
            pl.BlockSpec(w.shape, lambda i: (0, 0)),
            pl.BlockSpec((1, w.shape[0]), lambda i: (0, 0)),
        ],
        out_specs=pl.BlockSpec((BR, w.shape[0]), lambda i: (i, 0)),
        out_shape=jax.ShapeDtypeStruct((x.shape[0], w.shape[0]), jnp.float32),
    )(x, w, b.reshape(1, -1))


def _combine_body(p_ref, c_ref, o_ref):
    s = p_ref[0] + p_ref[1]
    cnt = c_ref[0, :, 0:1] + c_ref[1, :, 0:1]
    o_ref[...] = s * (1.0 / jnp.maximum(cnt, 1.0))


def _tc_combine(partials, cnts):
    return pl.pallas_call(
        _combine_body,
        grid=(GR,),
        in_specs=[
            pl.BlockSpec((NC, BR, H), lambda i: (0, i, 0)),
            pl.BlockSpec((NC, BR, 16), lambda i: (0, i, 0)),
        ],
        out_specs=pl.BlockSpec((BR, H), lambda i: (i, 0)),
        out_shape=jax.ShapeDtypeStruct((NE, H), jnp.float32),
    )(partials, cnts)


def _layer_body(beta, p_ref, c_ref, h0_ref, w_ref, o_ref):
    s = p_ref[0] + p_ref[1]
    cnt = c_ref[0, :, 0:1] + c_ref[1, :, 0:1]
    xv = s * (1.0 / jnp.maximum(cnt, 1.0))
    rn = jnp.sqrt(jnp.sum(xv * xv, axis=1, keepdims=True))
    xn = xv * jnp.where(rn > 0, 1.0 / rn, 0.0)
    xi = (1.0 - ALPHA) * xn + ALPHA * h0_ref[...]
    xw = lax.dot_general(xi, w_ref[...], (((1,), (1,)), ((), ())),
                         preferred_element_type=jnp.float32)
    o_ref[...] = jnp.maximum((1.0 - beta) * xi + beta * xw, 0.0)


def _final_body(beta, p_ref, c_ref, h0_ref, w_ref, wo_ref, bo_ref, o_ref):
    s = p_ref[0] + p_ref[1]
    cnt = c_ref[0, :, 0:1] + c_ref[1, :, 0:1]
    xv = s * (1.0 / jnp.maximum(cnt, 1.0))
    rn = jnp.sqrt(jnp.sum(xv * xv, axis=1, keepdims=True))
    xn = xv * jnp.where(rn > 0, 1.0 / rn, 0.0)
    xi = (1.0 - ALPHA) * xn + ALPHA * h0_ref[...]
    xw = lax.dot_general(xi, w_ref[...], (((1,), (1,)), ((), ())),
                         preferred_element_type=jnp.float32)
    h = jnp.maximum((1.0 - beta) * xi + beta * xw, 0.0)
    o_ref[...] = lax.dot_general(h, wo_ref[...], (((1,), (1,)), ((), ())),
                                 preferred_element_type=jnp.float32) + bo_ref[...]


def _tc_layer(partials, cnts, h0, w, beta):
    return pl.pallas_call(
        functools.partial(_layer_body, beta),
        grid=(GR,),
        in_specs=[
            pl.BlockSpec((NC, BR, H), lambda i: (0, i, 0)),
            pl.BlockSpec((NC, BR, 16), lambda i: (0, i, 0)),
            pl.BlockSpec((BR, H), lambda i: (i, 0)),
            pl.BlockSpec((H, H), lambda i: (0, 0)),
        ],
        out_specs=pl.BlockSpec((BR, H), lambda i: (i, 0)),
        out_shape=jax.ShapeDtypeStruct((N, H), jnp.float32),
    )(partials, cnts, h0, w)


def _tc_final(partials, cnts, h0, w, beta, wout, bout):
    return pl.pallas_call(
        functools.partial(_final_body, beta),
        grid=(GR,),
        in_specs=[
            pl.BlockSpec((NC, BR, H), lambda i: (0, i, 0)),
            pl.BlockSpec((NC, BR, 16), lambda i: (0, i, 0)),
            pl.BlockSpec((BR, H), lambda i: (i, 0)),
            pl.BlockSpec((H, H), lambda i: (0, 0)),
            pl.BlockSpec((C, H), lambda i: (0, 0)),
            pl.BlockSpec((1, C), lambda i: (0, 0)),
        ],
        out_specs=pl.BlockSpec((BR, C), lambda i: (i, 0)),
        out_shape=jax.ShapeDtypeStruct((N, C), jnp.float32),
    )(partials, cnts, h0, w, wout, bout.reshape(1, -1))


def kernel(x, edge_index, W0, b0, Ws, Wout, bout):
    vw = edge_index[0].reshape(NW, EPW)
    ew = edge_index[1].reshape(NW, EPW)
    vertex_m = vw[:, :FCH * B].reshape(NW, PH, PCH, B)
    edges_m = ew[:, :FCH * B].reshape(NW, PH, PCH, B)
    vertex_t = vw[:, FCH * B:]
    edges_t = ew[:, FCH * B:]
    zrows = jnp.zeros((STRIPE, H), jnp.float32)
    z16 = jnp.zeros((STRIPE, 16), jnp.float32)
    ones = jnp.ones((B, 16), jnp.float32)

    h = _tc_linear(x, W0, b0, relu=True)
    h0 = h
    ce, cv = _sc_count(edges_m, vertex_m, edges_t, vertex_t, z16, ones)
    for i in range(DEPTH):
        beta = math.log(0.5 / (i + 1) + 1.0)
        sep = _sc_agg(h, vertex_m, edges_m, vertex_t, edges_t, zrows)
        xe = _tc_combine(sep, ce)
        svp = _sc_agg(xe, edges_m, vertex_m, edges_t, vertex_t, zrows)
        if i < DEPTH - 1:
            h = _tc_layer(svp, cv, h0, Ws[i], beta)
        else:
            return _tc_final(svp, cv, h0, Ws[i], beta, Wout, bout)


# speedup vs baseline: 1.2571x; 1.0405x over previous
import functools
import math

import jax
import jax.numpy as jnp
from jax import lax
from jax.experimental import pallas as pl
from jax.experimental.pallas import tpu as pltpu
from jax.experimental.pallas import tpu_sc as plsc

N = 10000
E = 320000
NE = 10000
F = 128
H = 128
C = 128
DEPTH = 4
ALPHA = 0.1

NC = 2
NS = 16
NW = NC * NS
EPW = E // NW
B = 128
PH = 6
PCH = 13
FCH = PH * PCH
TB = EPW - FCH * B
NP = 10240
STRIPE = NP // NS

_mesh = plsc.VectorSubcoreMesh(core_axis_name="c", subcore_axis_name="s")


@functools.partial(
    pl.kernel,
    out_type=jax.ShapeDtypeStruct((NC, NP, H), jnp.float32),
    mesh=_mesh,
    scratch_types=[
        pltpu.VMEM((PCH, B), jnp.int32),
        pltpu.VMEM((PCH, B), jnp.int32),
        pltpu.VMEM((PCH, B), jnp.int32),
        pltpu.VMEM((PCH, B), jnp.int32),
        pltpu.VMEM((B,), jnp.int32),
        pltpu.VMEM((B,), jnp.int32),
        pltpu.VMEM((TB,), jnp.int32),
        pltpu.VMEM((TB,), jnp.int32),
        pltpu.VMEM((B, H), jnp.float32),
        pltpu.VMEM((B, H), jnp.float32),
        pltpu.VMEM_SHARED((NP, H), jnp.float32),
        pltpu.SemaphoreType.DMA,
        pltpu.SemaphoreType.DMA,
        pltpu.SemaphoreType.DMA,
        pltpu.SemaphoreType.DMA,
    ],
)
def _sc_agg(src, idxg, idxs, idxgt, idxst, zrows, out,
            igA, isA, igB, isB, s0, s1, tg, ts, r0, r1, acc, g0, g1, zs, isem):
    cid = lax.axis_index("c")
    sid = lax.axis_index("s")
    wid = sid * NC + cid
    pltpu.async_copy(zrows, acc.at[pl.ds(sid * STRIPE, STRIPE)], zs)
    pltpu.sync_copy(idxg.at[wid, 0], igA)
    pltpu.sync_copy(idxs.at[wid, 0], isA)
    pltpu.make_async_copy(zrows, acc.at[pl.ds(sid * STRIPE, STRIPE)], zs).wait()
    plsc.subcore_barrier()

    def stage(isv, c, dst):
        for k in range(B // 16):
            dst[pl.ds(16 * k, 16)] = isv[c, pl.ds(16 * k, 16)]

    for p in range(PH):
        igv, isv = (igA, isA) if p % 2 == 0 else (igB, isB)
        ngv, nsv = (igB, isB) if p % 2 == 0 else (igA, isA)
        stage(isv, 0, s0)
        pltpu.async_copy(src.at[igv.at[0]], r0, g0)
        if p + 1 < PH:
            pltpu.async_copy(idxg.at[wid, p + 1], ngv, isem)
            pltpu.async_copy(idxs.at[wid, p + 1], nsv, isem)

        def pair(g, carry2, igv=igv, isv=isv):
            c0 = 2 * g
            stage(isv, c0 + 1, s1)
            pltpu.async_copy(src.at[igv.at[c0 + 1]], r1, g1)
            pltpu.make_async_copy(src.at[igv.at[c0]], r0, g0).wait()
            pltpu.sync_copy(r0, acc.at[s0], add=True)
            stage(isv, c0 + 2, s0)
            pltpu.async_copy(src.at[igv.at[c0 + 2]], r0, g0)
            pltpu.make_async_copy(src.at[igv.at[c0 + 1]], r1, g1).wait()
            pltpu.sync_copy(r1, acc.at[s1], add=True)
            return carry2

        lax.fori_loop(0, (PCH - 1) // 2, pair, 0)
        pltpu.make_async_copy(src.at[igv.at[PCH - 1]], r0, g0).wait()
        pltpu.sync_copy(r0, acc.at[s0], add=True)
        if p + 1 < PH:
            pltpu.make_async_copy(idxg.at[wid, p + 1], ngv, isem).wait()
            pltpu.make_async_copy(idxs.at[wid, p + 1], nsv, isem).wait()

    pltpu.sync_copy(idxgt.at[wid], tg)
    pltpu.sync_copy(idxst.at[wid], ts)
    pltpu.async_copy(src.at[tg], r0.at[pl.ds(0, TB)], g0)
    pltpu.make_async_copy(src.at[tg], r0.at[pl.ds(0, TB)], g0).wait()
    pltpu.sync_copy(r0.at[pl.ds(0, TB)], acc.at[ts], add=True)

    plsc.subcore_barrier()
    pltpu.sync_copy(acc.at[pl.ds(sid * STRIPE, STRIPE)],
                    out.at[cid, pl.ds(sid * STRIPE, STRIPE)])


@functools.partial(
    pl.kernel,
    out_type=(jax.ShapeDtypeStruct((NC, NP, 16), jnp.float32),
              jax.ShapeDtypeStruct((NC, NP, 16), jnp.float32)),
    mesh=_mesh,
    scratch_types=[
        pltpu.VMEM((PCH, B), jnp.int32),
        pltpu.VMEM((PCH, B), jnp.int32),
        pltpu.VMEM((B,), jnp.int32),
        pltpu.VMEM((B,), jnp.int32),
        pltpu.VMEM((B,), jnp.int32),
        pltpu.VMEM((B,), jnp.int32),
        pltpu.VMEM((TB,), jnp.int32),
        pltpu.VMEM((TB,), jnp.int32),
        pltpu.VMEM((B, 16), jnp.float32),
        pltpu.VMEM_SHARED((NP, 16), jnp.float32),
        pltpu.VMEM_SHARED((NP, 16), jnp.float32),
        pltpu.SemaphoreType.DMA,
        pltpu.SemaphoreType.DMA,
        pltpu.SemaphoreType.DMA,
        pltpu.SemaphoreType.DMA,
    ],
)
def _sc_count(idxe, idxv, idxet, idxvt, z16, ones_hbm, oute, outv,
              idxe_v, idxv_v, se0, se1, sv0, sv1, te, tv, ones_v,
              acce, accv, ae0, av0, ae1, av1):
    cid = lax.axis_index("c")
    sid = lax.axis_index("s")
    wid = sid * NC + cid
    pltpu.sync_copy(ones_hbm, ones_v)
    pltpu.sync_copy(z16, acce.at[pl.ds(sid * STRIPE, STRIPE)])
    pltpu.sync_copy(z16, accv.at[pl.ds(sid * STRIPE, STRIPE)])
    plsc.subcore_barrier()

    def stage(ref, c, dst):
        for k in range(B // 16):
            dst[pl.ds(16 * k, 16)] = ref[c, pl.ds(16 * k, 16)]

    def issue(se, sv, semee, semv):
        pltpu.async_copy(ones_v, acce.at[se], semee, add=True)
        pltpu.async_copy(ones_v, accv.at[sv], semv, add=True)

    def drain(se, sv, semee, semv):
        pltpu.make_async_copy(ones_v, acce.at[se], semee).wait()
        pltpu.make_async_copy(ones_v, accv.at[sv], semv).wait()

    def phase(p, carry):
        pltpu.sync_copy(idxe.at[wid, p], idxe_v)
        pltpu.sync_copy(idxv.at[wid, p], idxv_v)
        stage(idxe_v, 0, se0)
        stage(idxv_v, 0, sv0)
        issue(se0, sv0, ae0, av0)
        stage(idxe_v, 1, se1)
        stage(idxv_v, 1, sv1)
        issue(se1, sv1, ae1, av1)

        def pair(g, carry2):
            c = 2 * g + 2
            drain(se0, sv0, ae0, av0)
            stage(idxe_v, c, se0)
            stage(idxv_v, c, sv0)
            issue(se0, sv0, ae0, av0)
            drain(se1, sv1, ae1, av1)
            stage(idxe_v, c + 1, se1)
            stage(idxv_v, c + 1, sv1)
            issue(se1, sv1, ae1, av1)
            return carry2

        lax.fori_loop(0, (PCH - 3) // 2, pair, 0)
        drain(se0, sv0, ae0, av0)
        stage(idxe_v, PCH - 1, se0)
        stage(idxv_v, PCH - 1, sv0)
        issue(se0, sv0, ae0, av0)
        drain(se0, sv0, ae0, av0)
        drain(se1, sv1, ae1, av1)
        return carry

    lax.fori_loop(0, PH, phase, 0)

    pltpu.sync_copy(idxet.at[wid], te)
    pltpu.sync_copy(idxvt.at[wid], tv)
    pltpu.sync_copy(ones_v.at[pl.ds(0, TB)], acce.at[te], add=True)
    pltpu.sync_copy(ones_v.at[pl.ds(0, TB)], accv.at[tv], add=True)

    plsc.subcore_barrier()
    pltpu.sync_copy(acce.at[pl.ds(sid * STRIPE, STRIPE)],
                    oute.at[cid, pl.ds(sid * STRIPE, STRIPE)])
    pltpu.sync_copy(accv.at[pl.ds(sid * STRIPE, STRIPE)],
                    outv.at[cid, pl.ds(sid * STRIPE, STRIPE)])


BR = 400
GR = N // BR


def _linear_body(relu, x_ref, w_ref, b_ref, o_ref):
    acc = lax.dot_general(x_ref[...], w_ref[...], (((1,), (1,)), ((), ())),
                          preferred_element_type=jnp.float32)
    acc = acc + b_ref[...]
    o_ref[...] = jnp.maximum(acc, 0.0) if relu else acc


def _tc_linear(x, w, b, relu):
    return pl.pallas_call(
        functools.partial(_linear_body, relu),
        grid=(GR,),
        in_specs=[
            pl.BlockSpec((BR, x.shape[1]), lambda i: (i, 0)),
            pl.BlockSpec(w.shape, lambda i: (0, 0)),
            pl.BlockSpec((1, w.shape[0]), lambda i: (0, 0)),
        ],
        out_specs=pl.BlockSpec((BR, w.shape[0]), lambda i: (i, 0)),
        out_shape=jax.ShapeDtypeStruct((x.shape[0], w.shape[0]), jnp.float32),
    )(x, w, b.reshape(1, -1))


def _combine_body(p_ref, c_ref, o_ref):
    s = p_ref[0] + p_ref[1]
    cnt = c_ref[0, :, 0:1] + c_ref[1, :, 0:1]
    o_ref[...] = s * (1.0 / jnp.maximum(cnt, 1.0))


def _tc_combine(partials, cnts):
    return pl.pallas_call(
        _combine_body,
        grid=(GR,),
        in_specs=[
            pl.BlockSpec((NC, BR, H), lambda i: (0, i, 0)),
            pl.BlockSpec((NC, BR, 16), lambda i: (0, i, 0)),
        ],
        out_specs=pl.BlockSpec((BR, H), lambda i: (i, 0)),
        out_shape=jax.ShapeDtypeStruct((NE, H), jnp.float32),
    )(partials, cnts)


def _layer_body(beta, p_ref, c_ref, h0_ref, w_ref, o_ref):
    s = p_ref[0] + p_ref[1]
    cnt = c_ref[0, :, 0:1] + c_ref[1, :, 0:1]
    xv = s * (1.0 / jnp.maximum(cnt, 1.0))
    rn = jnp.sqrt(jnp.sum(xv * xv, axis=1, keepdims=True))
    xn = xv * jnp.where(rn > 0, 1.0 / rn, 0.0)
    xi = (1.0 - ALPHA) * xn + ALPHA * h0_ref[...]
    xw = lax.dot_general(xi, w_ref[...], (((1,), (1,)), ((), ())),
                         preferred_element_type=jnp.float32)
    o_ref[...] = jnp.maximum((1.0 - beta) * xi + beta * xw, 0.0)


def _final_body(beta, p_ref, c_ref, h0_ref, w_ref, wo_ref, bo_ref, o_ref):
    s = p_ref[0] + p_ref[1]
    cnt = c_ref[0, :, 0:1] + c_ref[1, :, 0:1]
    xv = s * (1.0 / jnp.maximum(cnt, 1.0))
    rn = jnp.sqrt(jnp.sum(xv * xv, axis=1, keepdims=True))
    xn = xv * jnp.where(rn > 0, 1.0 / rn, 0.0)
    xi = (1.0 - ALPHA) * xn + ALPHA * h0_ref[...]
    xw = lax.dot_general(xi, w_ref[...], (((1,), (1,)), ((), ())),
                         preferred_element_type=jnp.float32)
    h = jnp.maximum((1.0 - beta) * xi + beta * xw, 0.0)
    o_ref[...] = lax.dot_general(h, wo_ref[...], (((1,), (1,)), ((), ())),
                                 preferred_element_type=jnp.float32) + bo_ref[...]


def _tc_layer(partials, cnts, h0, w, beta):
    return pl.pallas_call(
        functools.partial(_layer_body, beta),
        grid=(GR,),
        in_specs=[
            pl.BlockSpec((NC, BR, H), lambda i: (0, i, 0)),
            pl.BlockSpec((NC, BR, 16), lambda i: (0, i, 0)),
            pl.BlockSpec((BR, H), lambda i: (i, 0)),
            pl.BlockSpec((H, H), lambda i: (0, 0)),
        ],
        out_specs=pl.BlockSpec((BR, H), lambda i: (i, 0)),
        out_shape=jax.ShapeDtypeStruct((N, H), jnp.float32),
    )(partials, cnts, h0, w)


def _tc_final(partials, cnts, h0, w, beta, wout, bout):
    return pl.pallas_call(
        functools.partial(_final_body, beta),
        grid=(GR,),
        in_specs=[
            pl.BlockSpec((NC, BR, H), lambda i: (0, i, 0)),
            pl.BlockSpec((NC, BR, 16), lambda i: (0, i, 0)),
            pl.BlockSpec((BR, H), lambda i: (i, 0)),
            pl.BlockSpec((H, H), lambda i: (0, 0)),
            pl.BlockSpec((C, H), lambda i: (0, 0)),
            pl.BlockSpec((1, C), lambda i: (0, 0)),
        ],
        out_specs=pl.BlockSpec((BR, C), lambda i: (i, 0)),
        out_shape=jax.ShapeDtypeStruct((N, C), jnp.float32),
    )(partials, cnts, h0, w, wout, bout.reshape(1, -1))


def kernel(x, edge_index, W0, b0, Ws, Wout, bout):
    vw = edge_index[0].reshape(NW, EPW)
    ew = edge_index[1].reshape(NW, EPW)
    vertex_m = vw[:, :FCH * B].reshape(NW, PH, PCH, B)
    edges_m = ew[:, :FCH * B].reshape(NW, PH, PCH, B)
    vertex_t = vw[:, FCH * B:]
    edges_t = ew[:, FCH * B:]
    zrows = jnp.zeros((STRIPE, H), jnp.float32)
    z16 = jnp.zeros((STRIPE, 16), jnp.float32)
    ones = jnp.ones((B, 16), jnp.float32)

    h = _tc_linear(x, W0, b0, relu=True)
    h0 = h
    ce, cv = _sc_count(edges_m, vertex_m, edges_t, vertex_t, z16, ones)
    for i in range(DEPTH):
        beta = math.log(0.5 / (i + 1) + 1.0)
        sep = _sc_agg(h, vertex_m, edges_m, vertex_t, edges_t, zrows)
        xe = _tc_combine(sep, ce)
        svp = _sc_agg(xe, edges_m, vertex_m, edges_t, vertex_t, zrows)
        if i < DEPTH - 1:
            h = _tc_layer(svp, cv, h0, Ws[i], beta)
        else:
            return _tc_final(svp, cv, h0, Ws[i], beta, Wout, bout)


# speedup vs baseline: 1.2617x; 1.0037x over previous
import functools
import math

import jax
import jax.numpy as jnp
from jax import lax
from jax.experimental import pallas as pl
from jax.experimental.pallas import tpu as pltpu
from jax.experimental.pallas import tpu_sc as plsc

N = 10000
E = 320000
NE = 10000
F = 128
H = 128
C = 128
DEPTH = 4
ALPHA = 0.1

NC = 2
NS = 16
NW = NC * NS
EPW = E // NW
B = 128
PH = 6
PCH = 13
FCH = PH * PCH
TB = EPW - FCH * B
NP = 10240
STRIPE = NP // NS

_mesh = plsc.VectorSubcoreMesh(core_axis_name="c", subcore_axis_name="s")


@functools.partial(
    pl.kernel,
    out_type=jax.ShapeDtypeStruct((NC, NP, H), jnp.float32),
    mesh=_mesh,
    scratch_types=[
        pltpu.VMEM((PCH, B), jnp.int32),
        pltpu.VMEM((PCH, B), jnp.int32),
        pltpu.VMEM((PCH, B), jnp.int32),
        pltpu.VMEM((PCH, B), jnp.int32),
        pltpu.VMEM((B,), jnp.int32),
        pltpu.VMEM((B,), jnp.int32),
        pltpu.VMEM((TB,), jnp.int32),
        pltpu.VMEM((TB,), jnp.int32),
        pltpu.VMEM((B, H), jnp.float32),
        pltpu.VMEM((B, H), jnp.float32),
        pltpu.VMEM_SHARED((NP, H), jnp.float32),
        pltpu.SemaphoreType.DMA,
        pltpu.SemaphoreType.DMA,
        pltpu.SemaphoreType.DMA,
        pltpu.SemaphoreType.DMA,
    ],
)
def _sc_agg(src, idxg, idxs, idxgt, idxst, zrows, out,
            igA, isA, igB, isB, s0, s1, tg, ts, r0, r1, acc, g0, g1, zs, isem):
    cid = lax.axis_index("c")
    sid = lax.axis_index("s")
    wid = sid * NC + cid
    pltpu.async_copy(zrows, acc.at[pl.ds(sid * STRIPE, STRIPE)], zs)
    pltpu.sync_copy(idxg.at[wid, 0], igA)
    pltpu.sync_copy(idxs.at[wid, 0], isA)
    pltpu.make_async_copy(zrows, acc.at[pl.ds(sid * STRIPE, STRIPE)], zs).wait()
    plsc.subcore_barrier()

    def stage(isv, c, dst):
        for k in range(B // 16):
            dst[pl.ds(16 * k, 16)] = isv[c, pl.ds(16 * k, 16)]

    for p in range(PH):
        igv, isv = (igA, isA) if p % 2 == 0 else (igB, isB)
        ngv, nsv = (igB, isB) if p % 2 == 0 else (igA, isA)
        stage(isv, 0, s0)
        pltpu.async_copy(src.at[igv.at[0]], r0, g0)
        if p + 1 < PH:
            pltpu.async_copy(idxg.at[wid, p + 1], ngv, isem)
            pltpu.async_copy(idxs.at[wid, p + 1], nsv, isem)

        def pair(g, carry2, igv=igv, isv=isv):
            c0 = 2 * g
            stage(isv, c0 + 1, s1)
            pltpu.async_copy(src.at[igv.at[c0 + 1]], r1, g1)
            pltpu.make_async_copy(src.at[igv.at[c0]], r0, g0).wait()
            pltpu.sync_copy(r0, acc.at[s0], add=True)
            stage(isv, c0 + 2, s0)
            pltpu.async_copy(src.at[igv.at[c0 + 2]], r0, g0)
            pltpu.make_async_copy(src.at[igv.at[c0 + 1]], r1, g1).wait()
            pltpu.sync_copy(r1, acc.at[s1], add=True)
            return carry2

        lax.fori_loop(0, (PCH - 1) // 2, pair, 0)
        pltpu.make_async_copy(src.at[igv.at[PCH - 1]], r0, g0).wait()
        pltpu.sync_copy(r0, acc.at[s0], add=True)
        if p + 1 < PH:
            pltpu.make_async_copy(idxg.at[wid, p + 1], ngv, isem).wait()
            pltpu.make_async_copy(idxs.at[wid, p + 1], nsv, isem).wait()

    pltpu.sync_copy(idxgt.at[wid], tg)
    pltpu.sync_copy(idxst.at[wid], ts)
    pltpu.async_copy(src.at[tg], r0.at[pl.ds(0, TB)], g0)
    pltpu.make_async_copy(src.at[tg], r0.at[pl.ds(0, TB)], g0).wait()
    pltpu.sync_copy(r0.at[pl.ds(0, TB)], acc.at[ts], add=True)

    plsc.subcore_barrier()
    pltpu.sync_copy(acc.at[pl.ds(sid * STRIPE, STRIPE)],
                    out.at[cid, pl.ds(sid * STRIPE, STRIPE)])


@functools.partial(
    pl.kernel,
    out_type=(jax.ShapeDtypeStruct((NC, NP, 16), jnp.float32),
              jax.ShapeDtypeStruct((NC, NP, 16), jnp.float32)),
    mesh=_mesh,
    scratch_types=[
        pltpu.VMEM((PCH, B), jnp.int32),
        pltpu.VMEM((PCH, B), jnp.int32),
        pltpu.VMEM((PCH, B), jnp.int32),
        pltpu.VMEM((PCH, B), jnp.int32),
        pltpu.VMEM((B,), jnp.int32),
        pltpu.VMEM((B,), jnp.int32),
        pltpu.VMEM((B,), jnp.int32),
        pltpu.VMEM((B,), jnp.int32),
        pltpu.VMEM((TB,), jnp.int32),
        pltpu.VMEM((TB,), jnp.int32),
        pltpu.VMEM((B, 16), jnp.float32),
        pltpu.VMEM_SHARED((NP, 16), jnp.float32),
        pltpu.VMEM_SHARED((NP, 16), jnp.float32),
        pltpu.SemaphoreType.DMA,
        pltpu.SemaphoreType.DMA,
        pltpu.SemaphoreType.DMA,
        pltpu.SemaphoreType.DMA,
        pltpu.SemaphoreType.DMA,
    ],
)
def _sc_count(idxe, idxv, idxet, idxvt, z16, ones_hbm, oute, outv,
              ieA, ivA, ieB, ivB, se0, se1, sv0, sv1, te, tv, ones_v,
              acce, accv, ae0, av0, ae1, av1, isem):
    cid = lax.axis_index("c")
    sid = lax.axis_index("s")
    wid = sid * NC + cid
    pltpu.async_copy(z16, acce.at[pl.ds(sid * STRIPE, STRIPE)], ae0)
    pltpu.async_copy(z16, accv.at[pl.ds(sid * STRIPE, STRIPE)], av0)
    pltpu.sync_copy(ones_hbm, ones_v)
    pltpu.sync_copy(idxe.at[wid, 0], ieA)
    pltpu.sync_copy(idxv.at[wid, 0], ivA)
    pltpu.make_async_copy(z16, acce.at[pl.ds(sid * STRIPE, STRIPE)], ae0).wait()
    pltpu.make_async_copy(z16, accv.at[pl.ds(sid * STRIPE, STRIPE)], av0).wait()
    plsc.subcore_barrier()

    def stage(ref, c, dst):
        for k in range(B // 16):
            dst[pl.ds(16 * k, 16)] = ref[c, pl.ds(16 * k, 16)]

    def issue(se, sv, semee, semv):
        pltpu.async_copy(ones_v, acce.at[se], semee, add=True)
        pltpu.async_copy(ones_v, accv.at[sv], semv, add=True)

    def drain(se, sv, semee, semv):
        pltpu.make_async_copy(ones_v, acce.at[se], semee).wait()
        pltpu.make_async_copy(ones_v, accv.at[sv], semv).wait()

    for p in range(PH):
        iev, ivv = (ieA, ivA) if p % 2 == 0 else (ieB, ivB)
        nev, nvv = (ieB, ivB) if p % 2 == 0 else (ieA, ivA)
        stage(iev, 0, se0)
        stage(ivv, 0, sv0)
        issue(se0, sv0, ae0, av0)
        stage(iev, 1, se1)
        stage(ivv, 1, sv1)
        issue(se1, sv1, ae1, av1)
        if p + 1 < PH:
            pltpu.async_copy(idxe.at[wid, p + 1], nev, isem)
            pltpu.async_copy(idxv.at[wid, p + 1], nvv, isem)

        def pair(g, carry2, iev=iev, ivv=ivv):
            c = 2 * g + 2
            drain(se0, sv0, ae0, av0)
            stage(iev, c, se0)
            stage(ivv, c, sv0)
            issue(se0, sv0, ae0, av0)
            drain(se1, sv1, ae1, av1)
            stage(iev, c + 1, se1)
            stage(ivv, c + 1, sv1)
            issue(se1, sv1, ae1, av1)
            return carry2

        lax.fori_loop(0, (PCH - 3) // 2, pair, 0)
        drain(se0, sv0, ae0, av0)
        stage(iev, PCH - 1, se0)
        stage(ivv, PCH - 1, sv0)
        issue(se0, sv0, ae0, av0)
        drain(se0, sv0, ae0, av0)
        drain(se1, sv1, ae1, av1)
        if p + 1 < PH:
            pltpu.make_async_copy(idxe.at[wid, p + 1], nev, isem).wait()
            pltpu.make_async_copy(idxv.at[wid, p + 1], nvv, isem).wait()

    pltpu.sync_copy(idxet.at[wid], te)
    pltpu.sync_copy(idxvt.at[wid], tv)
    pltpu.sync_copy(ones_v.at[pl.ds(0, TB)], acce.at[te], add=True)
    pltpu.sync_copy(ones_v.at[pl.ds(0, TB)], accv.at[tv], add=True)

    plsc.subcore_barrier()
    pltpu.sync_copy(acce.at[pl.ds(sid * STRIPE, STRIPE)],
                    oute.at[cid, pl.ds(sid * STRIPE, STRIPE)])
    pltpu.sync_copy(accv.at[pl.ds(sid * STRIPE, STRIPE)],
                    outv.at[cid, pl.ds(sid * STRIPE, STRIPE)])


BR = 400
GR = N // BR


def _linear_body(relu, x_ref, w_ref, b_ref, o_ref):
    acc = lax.dot_general(x_ref[...], w_ref[...], (((1,), (1,)), ((), ())),
                          preferred_element_type=jnp.float32)
    acc = acc + b_ref[...]
    o_ref[...] = jnp.maximum(acc, 0.0) if relu else acc


def _tc_linear(x, w, b, relu):
    return pl.pallas_call(
        functools.partial(_linear_body, relu),
        grid=(GR,),
        in_specs=[
            pl.BlockSpec((BR, x.shape[1]), lambda i: (i, 0)),
            pl.BlockSpec(w.shape, lambda i: (0, 0)),
            pl.BlockSpec((1, w.shape[0]), lambda i: (0, 0)),
        ],
        out_specs=pl.BlockSpec((BR, w.shape[0]), lambda i: (i, 0)),
        out_shape=jax.ShapeDtypeStruct((x.shape[0], w.shape[0]), jnp.float32),
    )(x, w, b.reshape(1, -1))


def _combine_body(p_ref, c_ref, o_ref):
    s = p_ref[0] + p_ref[1]
    cnt = c_ref[0, :, 0:1] + c_ref[1, :, 0:1]
    o_ref[...] = s * (1.0 / jnp.maximum(cnt, 1.0))


def _tc_combine(partials, cnts):
    return pl.pallas_call(
        _combine_body,
        grid=(GR,),
        in_specs=[
            pl.BlockSpec((NC, BR, H), lambda i: (0, i, 0)),
            pl.BlockSpec((NC, BR, 16), lambda i: (0, i, 0)),
        ],
        out_specs=pl.BlockSpec((BR, H), lambda i: (i, 0)),
        out_shape=jax.ShapeDtypeStruct((NE, H), jnp.float32),
    )(partials, cnts)


def _layer_body(beta, p_ref, c_ref, h0_ref, w_ref, o_ref):
    s = p_ref[0] + p_ref[1]
    cnt = c_ref[0, :, 0:1] + c_ref[1, :, 0:1]
    xv = s * (1.0 / jnp.maximum(cnt, 1.0))
    rn = jnp.sqrt(jnp.sum(xv * xv, axis=1, keepdims=True))
    xn = xv * jnp.where(rn > 0, 1.0 / rn, 0.0)
    xi = (1.0 - ALPHA) * xn + ALPHA * h0_ref[...]
    xw = lax.dot_general(xi, w_ref[...], (((1,), (1,)), ((), ())),
                         preferred_element_type=jnp.float32)
    o_ref[...] = jnp.maximum((1.0 - beta) * xi + beta * xw, 0.0)


def _final_body(beta, p_ref, c_ref, h0_ref, w_ref, wo_ref, bo_ref, o_ref):
    s = p_ref[0] + p_ref[1]
    cnt = c_ref[0, :, 0:1] + c_ref[1, :, 0:1]
    xv = s * (1.0 / jnp.maximum(cnt, 1.0))
    rn = jnp.sqrt(jnp.sum(xv * xv, axis=1, keepdims=True))
    xn = xv * jnp.where(rn > 0, 1.0 / rn, 0.0)
    xi = (1.0 - ALPHA) * xn + ALPHA * h0_ref[...]
    xw = lax.dot_general(xi, w_ref[...], (((1,), (1,)), ((), ())),
                         preferred_element_type=jnp.float32)
    h = jnp.maximum((1.0 - beta) * xi + beta * xw, 0.0)
    o_ref[...] = lax.dot_general(h, wo_ref[...], (((1,), (1,)), ((), ())),
                                 preferred_element_type=jnp.float32) + bo_ref[...]


def _tc_layer(partials, cnts, h0, w, beta):
    return pl.pallas_call(
        functools.partial(_layer_body, beta),
        grid=(GR,),
        in_specs=[
            pl.BlockSpec((NC, BR, H), lambda i: (0, i, 0)),
            pl.BlockSpec((NC, BR, 16), lambda i: (0, i, 0)),
            pl.BlockSpec((BR, H), lambda i: (i, 0)),
            pl.BlockSpec((H, H), lambda i: (0, 0)),
        ],
        out_specs=pl.BlockSpec((BR, H), lambda i: (i, 0)),
        out_shape=jax.ShapeDtypeStruct((N, H), jnp.float32),
    )(partials, cnts, h0, w)


def _tc_final(partials, cnts, h0, w, beta, wout, bout):
    return pl.pallas_call(
        functools.partial(_final_body, beta),
        grid=(GR,),
        in_specs=[
            pl.BlockSpec((NC, BR, H), lambda i: (0, i, 0)),
            pl.BlockSpec((NC, BR, 16), lambda i: (0, i, 0)),
            pl.BlockSpec((BR, H), lambda i: (i, 0)),
            pl.BlockSpec((H, H), lambda i: (0, 0)),
            pl.BlockSpec((C, H), lambda i: (0, 0)),
            pl.BlockSpec((1, C), lambda i: (0, 0)),
        ],
        out_specs=pl.BlockSpec((BR, C), lambda i: (i, 0)),
        out_shape=jax.ShapeDtypeStruct((N, C), jnp.float32),
    )(partials, cnts, h0, w, wout, bout.reshape(1, -1))


def kernel(x, edge_index, W0, b0, Ws, Wout, bout):
    vw = edge_index[0].reshape(NW, EPW)
    ew = edge_index[1].reshape(NW, EPW)
    vertex_m = vw[:, :FCH * B].reshape(NW, PH, PCH, B)
    edges_m = ew[:, :FCH * B].reshape(NW, PH, PCH, B)
    vertex_t = vw[:, FCH * B:]
    edges_t = ew[:, FCH * B:]
    zrows = jnp.zeros((STRIPE, H), jnp.float32)
    z16 = jnp.zeros((STRIPE, 16), jnp.float32)
    ones = jnp.ones((B, 16), jnp.float32)

    h = _tc_linear(x, W0, b0, relu=True)
    h0 = h
    ce, cv = _sc_count(edges_m, vertex_m, edges_t, vertex_t, z16, ones)
    for i in range(DEPTH):
        beta = math.log(0.5 / (i + 1) + 1.0)
        sep = _sc_agg(h, vertex_m, edges_m, vertex_t, edges_t, zrows)
        xe = _tc_combine(sep, ce)
        svp = _sc_agg(xe, edges_m, vertex_m, edges_t, vertex_t, zrows)
        if i < DEPTH - 1:
            h = _tc_layer(svp, cv, h0, Ws[i], beta)
        else:
            return _tc_final(svp, cv, h0, Ws[i], beta, Wout, bout)


# speedup vs baseline: 1.3518x; 1.0713x over previous
import functools
import math

import jax
import jax.numpy as jnp
from jax import lax
from jax.experimental import pallas as pl
from jax.experimental.pallas import tpu as pltpu
from jax.experimental.pallas import tpu_sc as plsc

N = 10000
E = 320000
NE = 10000
F = 128
H = 128
C = 128
DEPTH = 4
ALPHA = 0.1

NC = 2
NS = 16
NW = NC * NS
EPW = E // NW
B = 128
PH = 6
PCH = 13
FCH = PH * PCH
TB = EPW - FCH * B
NP = 10240
STRIPE = NP // NS

_mesh = plsc.VectorSubcoreMesh(core_axis_name="c", subcore_axis_name="s")


@functools.partial(
    pl.kernel,
    out_type=jax.ShapeDtypeStruct((NC, NP, H), jnp.float32),
    mesh=_mesh,
    scratch_types=[
        pltpu.VMEM((PCH, B), jnp.int32),
        pltpu.VMEM((PCH, B), jnp.int32),
        pltpu.VMEM((PCH, B), jnp.int32),
        pltpu.VMEM((PCH, B), jnp.int32),
        pltpu.VMEM((B,), jnp.int32),
        pltpu.VMEM((B,), jnp.int32),
        pltpu.VMEM((TB,), jnp.int32),
        pltpu.VMEM((TB,), jnp.int32),
        pltpu.VMEM((B, H), jnp.float32),
        pltpu.VMEM((B, H), jnp.float32),
        pltpu.VMEM_SHARED((NP, H), jnp.float32),
        pltpu.SemaphoreType.DMA,
        pltpu.SemaphoreType.DMA,
        pltpu.SemaphoreType.DMA,
        pltpu.SemaphoreType.DMA,
    ],
)
def _sc_agg(src, idxg, idxs, idxgt, idxst, zrows, out,
            igA, isA, igB, isB, s0, s1, tg, ts, r0, r1, acc, g0, g1, zs, isem):
    cid = lax.axis_index("c")
    sid = lax.axis_index("s")
    wid = sid * NC + cid
    pltpu.async_copy(zrows, acc.at[pl.ds(sid * STRIPE, STRIPE)], zs)
    pltpu.sync_copy(idxg.at[wid, 0], igA)
    pltpu.sync_copy(idxs.at[wid, 0], isA)

    def stage(isv, c, dst):
        for k in range(B // 16):
            dst[pl.ds(16 * k, 16)] = isv[c, pl.ds(16 * k, 16)]

    stage(isA, 0, s0)
    pltpu.async_copy(src.at[igA.at[0]], r0, g0)
    pltpu.make_async_copy(zrows, acc.at[pl.ds(sid * STRIPE, STRIPE)], zs).wait()
    plsc.subcore_barrier()

    for p in range(PH):
        igv, isv = (igA, isA) if p % 2 == 0 else (igB, isB)
        ngv, nsv = (igB, isB) if p % 2 == 0 else (igA, isA)
        rX, sX, gX = (r0, s0, g0) if p % 2 == 0 else (r1, s1, g1)
        rY, sY, gY = (r1, s1, g1) if p % 2 == 0 else (r0, s0, g0)
        if p + 1 < PH:
            pltpu.async_copy(idxg.at[wid, p + 1], ngv, isem)
            pltpu.async_copy(idxs.at[wid, p + 1], nsv, isem)
        else:
            pltpu.async_copy(idxgt.at[wid], tg, isem)
            pltpu.async_copy(idxst.at[wid], ts, isem)

        def pair(g, carry2, igv=igv, isv=isv, rX=rX, sX=sX, gX=gX,
                 rY=rY, sY=sY, gY=gY):
            c0 = 2 * g
            stage(isv, c0 + 1, sY)
            pltpu.async_copy(src.at[igv.at[c0 + 1]], rY, gY)
            pltpu.make_async_copy(src.at[igv.at[c0]], rX, gX).wait()
            pltpu.sync_copy(rX, acc.at[sX], add=True)
            stage(isv, c0 + 2, sX)
            pltpu.async_copy(src.at[igv.at[c0 + 2]], rX, gX)
            pltpu.make_async_copy(src.at[igv.at[c0 + 1]], rY, gY).wait()
            pltpu.sync_copy(rY, acc.at[sY], add=True)
            return carry2

        lax.fori_loop(0, (PCH - 3) // 2, pair, 0)
        stage(isv, PCH - 2, sY)
        pltpu.async_copy(src.at[igv.at[PCH - 2]], rY, gY)
        pltpu.make_async_copy(src.at[igv.at[PCH - 3]], rX, gX).wait()
        pltpu.sync_copy(rX, acc.at[sX], add=True)
        stage(isv, PCH - 1, sX)
        pltpu.async_copy(src.at[igv.at[PCH - 1]], rX, gX)
        pltpu.make_async_copy(src.at[igv.at[PCH - 2]], rY, gY).wait()
        pltpu.sync_copy(rY, acc.at[sY], add=True)
        if p + 1 < PH:
            pltpu.make_async_copy(idxg.at[wid, p + 1], ngv, isem).wait()
            pltpu.make_async_copy(idxs.at[wid, p + 1], nsv, isem).wait()
            stage(nsv, 0, sY)
            pltpu.async_copy(src.at[ngv.at[0]], rY, gY)
        else:
            pltpu.make_async_copy(idxgt.at[wid], tg, isem).wait()
            pltpu.make_async_copy(idxst.at[wid], ts, isem).wait()
            pltpu.async_copy(src.at[tg], rY.at[pl.ds(0, TB)], gY)
        pltpu.make_async_copy(src.at[igv.at[PCH - 1]], rX, gX).wait()
        pltpu.sync_copy(rX, acc.at[sX], add=True)

    rT = r1 if (PH - 1) % 2 == 0 else r0
    gT = g1 if (PH - 1) % 2 == 0 else g0
    pltpu.make_async_copy(src.at[tg], rT.at[pl.ds(0, TB)], gT).wait()
    pltpu.sync_copy(rT.at[pl.ds(0, TB)], acc.at[ts], add=True)

    plsc.subcore_barrier()
    pltpu.sync_copy(acc.at[pl.ds(sid * STRIPE, STRIPE)],
                    out.at[cid, pl.ds(sid * STRIPE, STRIPE)])


@functools.partial(
    pl.kernel,
    out_type=(jax.ShapeDtypeStruct((NC, NP, 16), jnp.float32),
              jax.ShapeDtypeStruct((NC, NP, 16), jnp.float32)),
    mesh=_mesh,
    scratch_types=[
        pltpu.VMEM((PCH, B), jnp.int32),
        pltpu.VMEM((PCH, B), jnp.int32),
        pltpu.VMEM((PCH, B), jnp.int32),
        pltpu.VMEM((PCH, B), jnp.int32),
        pltpu.VMEM((B,), jnp.int32),
        pltpu.VMEM((B,), jnp.int32),
        pltpu.VMEM((B,), jnp.int32),
        pltpu.VMEM((B,), jnp.int32),
        pltpu.VMEM((TB,), jnp.int32),
        pltpu.VMEM((TB,), jnp.int32),
        pltpu.VMEM((B, 16), jnp.float32),
        pltpu.VMEM_SHARED((NP, 16), jnp.float32),
        pltpu.VMEM_SHARED((NP, 16), jnp.float32),
        pltpu.SemaphoreType.DMA,
        pltpu.SemaphoreType.DMA,
        pltpu.SemaphoreType.DMA,
        pltpu.SemaphoreType.DMA,
        pltpu.SemaphoreType.DMA,
    ],
)
def _sc_count(idxe, idxv, idxet, idxvt, z16, ones_hbm, oute, outv,
              ieA, ivA, ieB, ivB, se0, se1, sv0, sv1, te, tv, ones_v,
              acce, accv, ae0, av0, ae1, av1, isem):
    cid = lax.axis_index("c")
    sid = lax.axis_index("s")
    wid = sid * NC + cid
    pltpu.async_copy(z16, acce.at[pl.ds(sid * STRIPE, STRIPE)], ae0)
    pltpu.async_copy(z16, accv.at[pl.ds(sid * STRIPE, STRIPE)], av0)
    pltpu.sync_copy(ones_hbm, ones_v)
    pltpu.sync_copy(idxe.at[wid, 0], ieA)
    pltpu.sync_copy(idxv.at[wid, 0], ivA)
    pltpu.make_async_copy(z16, acce.at[pl.ds(sid * STRIPE, STRIPE)], ae0).wait()
    pltpu.make_async_copy(z16, accv.at[pl.ds(sid * STRIPE, STRIPE)], av0).wait()
    plsc.subcore_barrier()

    def stage(ref, c, dst):
        for k in range(B // 16):
            dst[pl.ds(16 * k, 16)] = ref[c, pl.ds(16 * k, 16)]

    def issue(se, sv, semee, semv):
        pltpu.async_copy(ones_v, acce.at[se], semee, add=True)
        pltpu.async_copy(ones_v, accv.at[sv], semv, add=True)

    def drain(se, sv, semee, semv):
        pltpu.make_async_copy(ones_v, acce.at[se], semee).wait()
        pltpu.make_async_copy(ones_v, accv.at[sv], semv).wait()

    for p in range(PH):
        iev, ivv = (ieA, ivA) if p % 2 == 0 else (ieB, ivB)
        nev, nvv = (ieB, ivB) if p % 2 == 0 else (ieA, ivA)
        stage(iev, 0, se0)
        stage(ivv, 0, sv0)
        issue(se0, sv0, ae0, av0)
        stage(iev, 1, se1)
        stage(ivv, 1, sv1)
        issue(se1, sv1, ae1, av1)
        if p + 1 < PH:
            pltpu.async_copy(idxe.at[wid, p + 1], nev, isem)
            pltpu.async_copy(idxv.at[wid, p + 1], nvv, isem)

        def pair(g, carry2, iev=iev, ivv=ivv):
            c = 2 * g + 2
            drain(se0, sv0, ae0, av0)
            stage(iev, c, se0)
            stage(ivv, c, sv0)
            issue(se0, sv0, ae0, av0)
            drain(se1, sv1, ae1, av1)
            stage(iev, c + 1, se1)
            stage(ivv, c + 1, sv1)
            issue(se1, sv1, ae1, av1)
            return carry2

        lax.fori_loop(0, (PCH - 3) // 2, pair, 0)
        drain(se0, sv0, ae0, av0)
        stage(iev, PCH - 1, se0)
        stage(ivv, PCH - 1, sv0)
        issue(se0, sv0, ae0, av0)
        drain(se0, sv0, ae0, av0)
        drain(se1, sv1, ae1, av1)
        if p + 1 < PH:
            pltpu.make_async_copy(idxe.at[wid, p + 1], nev, isem).wait()
            pltpu.make_async_copy(idxv.at[wid, p + 1], nvv, isem).wait()

    pltpu.sync_copy(idxet.at[wid], te)
    pltpu.sync_copy(idxvt.at[wid], tv)
    pltpu.sync_copy(ones_v.at[pl.ds(0, TB)], acce.at[te], add=True)
    pltpu.sync_copy(ones_v.at[pl.ds(0, TB)], accv.at[tv], add=True)

    plsc.subcore_barrier()
    pltpu.sync_copy(acce.at[pl.ds(sid * STRIPE, STRIPE)],
                    oute.at[cid, pl.ds(sid * STRIPE, STRIPE)])
    pltpu.sync_copy(accv.at[pl.ds(sid * STRIPE, STRIPE)],
                    outv.at[cid, pl.ds(sid * STRIPE, STRIPE)])


BR = 400
GR = N // BR


def _linear_body(relu, x_ref, w_ref, b_ref, o_ref):
    acc = lax.dot_general(x_ref[...], w_ref[...], (((1,), (1,)), ((), ())),
                          preferred_element_type=jnp.float32)
    acc = acc + b_ref[...]
    o_ref[...] = jnp.maximum(acc, 0.0) if relu else acc


def _tc_linear(x, w, b, relu):
    return pl.pallas_call(
        functools.partial(_linear_body, relu),
        grid=(GR,),
        in_specs=[
            pl.BlockSpec((BR, x.shape[1]), lambda i: (i, 0)),
            pl.BlockSpec(w.shape, lambda i: (0, 0)),
            pl.BlockSpec((1, w.shape[0]), lambda i: (0, 0)),
        ],
        out_specs=pl.BlockSpec((BR, w.shape[0]), lambda i: (i, 0)),
        out_shape=jax.ShapeDtypeStruct((x.shape[0], w.shape[0]), jnp.float32),
    )(x, w, b.reshape(1, -1))


def _combine_body(p_ref, c_ref, o_ref):
    s = p_ref[0] + p_ref[1]
    cnt = c_ref[0, :, 0:1] + c_ref[1, :, 0:1]
    o_ref[...] = s * (1.0 / jnp.maximum(cnt, 1.0))


def _tc_combine(partials, cnts):
    return pl.pallas_call(
        _combine_body,
        grid=(GR,),
        in_specs=[
            pl.BlockSpec((NC, BR, H), lambda i: (0, i, 0)),
            pl.BlockSpec((NC, BR, 16), lambda i: (0, i, 0)),
        ],
        out_specs=pl.BlockSpec((BR, H), lambda i: (i, 0)),
        out_shape=jax.ShapeDtypeStruct((NE, H), jnp.float32),
    )(partials, cnts)


def _layer_body(beta, p_ref, c_ref, h0_ref, w_ref, o_ref):
    s = p_ref[0] + p_ref[1]
    cnt = c_ref[0, :, 0:1] + c_ref[1, :, 0:1]
    xv = s * (1.0 / jnp.maximum(cnt, 1.0))
    rn = jnp.sqrt(jnp.sum(xv * xv, axis=1, keepdims=True))
    xn = xv * jnp.where(rn > 0, 1.0 / rn, 0.0)
    xi = (1.0 - ALPHA) * xn + ALPHA * h0_ref[...]
    xw = lax.dot_general(xi, w_ref[...], (((1,), (1,)), ((), ())),
                         preferred_element_type=jnp.float32)
    o_ref[...] = jnp.maximum((1.0 - beta) * xi + beta * xw, 0.0)


def _final_body(beta, p_ref, c_ref, h0_ref, w_ref, wo_ref, bo_ref, o_ref):
    s = p_ref[0] + p_ref[1]
    cnt = c_ref[0, :, 0:1] + c_ref[1, :, 0:1]
    xv = s * (1.0 / jnp.maximum(cnt, 1.0))
    rn = jnp.sqrt(jnp.sum(xv * xv, axis=1, keepdims=True))
    xn = xv * jnp.where(rn > 0, 1.0 / rn, 0.0)
    xi = (1.0 - ALPHA) * xn + ALPHA * h0_ref[...]
    xw = lax.dot_general(xi, w_ref[...], (((1,), (1,)), ((), ())),
                         preferred_element_type=jnp.float32)
    h = jnp.maximum((1.0 - beta) * xi + beta * xw, 0.0)
    o_ref[...] = lax.dot_general(h, wo_ref[...], (((1,), (1,)), ((), ())),
                                 preferred_element_type=jnp.float32) + bo_ref[...]


def _tc_layer(partials, cnts, h0, w, beta):
    return pl.pallas_call(
        functools.partial(_layer_body, beta),
        grid=(GR,),
        in_specs=[
            pl.BlockSpec((NC, BR, H), lambda i: (0, i, 0)),
            pl.BlockSpec((NC, BR, 16), lambda i: (0, i, 0)),
            pl.BlockSpec((BR, H), lambda i: (i, 0)),
            pl.BlockSpec((H, H), lambda i: (0, 0)),
        ],
        out_specs=pl.BlockSpec((BR, H), lambda i: (i, 0)),
        out_shape=jax.ShapeDtypeStruct((N, H), jnp.float32),
    )(partials, cnts, h0, w)


def _tc_final(partials, cnts, h0, w, beta, wout, bout):
    return pl.pallas_call(
        functools.partial(_final_body, beta),
        grid=(GR,),
        in_specs=[
            pl.BlockSpec((NC, BR, H), lambda i: (0, i, 0)),
            pl.BlockSpec((NC, BR, 16), lambda i: (0, i, 0)),
            pl.BlockSpec((BR, H), lambda i: (i, 0)),
            pl.BlockSpec((H, H), lambda i: (0, 0)),
            pl.BlockSpec((C, H), lambda i: (0, 0)),
            pl.BlockSpec((1, C), lambda i: (0, 0)),
        ],
        out_specs=pl.BlockSpec((BR, C), lambda i: (i, 0)),
        out_shape=jax.ShapeDtypeStruct((N, C), jnp.float32),
    )(partials, cnts, h0, w, wout, bout.reshape(1, -1))


def kernel(x, edge_index, W0, b0, Ws, Wout, bout):
    vw = edge_index[0].reshape(NW, EPW)
    ew = edge_index[1].reshape(NW, EPW)
    vertex_m = vw[:, :FCH * B].reshape(NW, PH, PCH, B)
    edges_m = ew[:, :FCH * B].reshape(NW, PH, PCH, B)
    vertex_t = vw[:, FCH * B:]
    edges_t = ew[:, FCH * B:]
    zrows = jnp.zeros((STRIPE, H), jnp.float32)
    z16 = jnp.zeros((STRIPE, 16), jnp.float32)
    ones = jnp.ones((B, 16), jnp.float32)

    h = _tc_linear(x, W0, b0, relu=True)
    h0 = h
    ce, cv = _sc_count(edges_m, vertex_m, edges_t, vertex_t, z16, ones)
    for i in range(DEPTH):
        beta = math.log(0.5 / (i + 1) + 1.0)
        sep = _sc_agg(h, vertex_m, edges_m, vertex_t, edges_t, zrows)
        xe = _tc_combine(sep, ce)
        svp = _sc_agg(xe, edges_m, vertex_m, edges_t, vertex_t, zrows)
        if i < DEPTH - 1:
            h = _tc_layer(svp, cv, h0, Ws[i], beta)
        else:
            return _tc_final(svp, cv, h0, Ws[i], beta, Wout, bout)


# speedup vs baseline: 1.4345x; 1.0612x over previous
import functools
import math

import jax
import jax.numpy as jnp
from jax import lax
from jax.experimental import pallas as pl
from jax.experimental.pallas import tpu as pltpu
from jax.experimental.pallas import tpu_sc as plsc

N = 10000
E = 320000
NE = 10000
F = 128
H = 128
C = 128
DEPTH = 4
ALPHA = 0.1

NC = 2
NS = 16
NW = NC * NS
EPW = E // NW
B = 128
PH = 6
PCH = 13
FCH = PH * PCH
TB = EPW - FCH * B
NP = 10240
STRIPE = NP // NS

_mesh = plsc.VectorSubcoreMesh(core_axis_name="c", subcore_axis_name="s")


@functools.partial(
    pl.kernel,
    out_type=jax.ShapeDtypeStruct((NC, NP, H), jnp.float32),
    mesh=_mesh,
    scratch_types=[
        pltpu.VMEM((PCH, B), jnp.int32),
        pltpu.VMEM((PCH, B), jnp.int32),
        pltpu.VMEM((PCH, B), jnp.int32),
        pltpu.VMEM((PCH, B), jnp.int32),
        pltpu.VMEM((B,), jnp.int32),
        pltpu.VMEM((B,), jnp.int32),
        pltpu.VMEM((TB,), jnp.int32),
        pltpu.VMEM((TB,), jnp.int32),
        pltpu.VMEM((B, H), jnp.float32),
        pltpu.VMEM((B, H), jnp.float32),
        pltpu.VMEM_SHARED((NP, H), jnp.float32),
        pltpu.SemaphoreType.DMA,
        pltpu.SemaphoreType.DMA,
        pltpu.SemaphoreType.DMA,
        pltpu.SemaphoreType.DMA,
    ],
)
def _sc_agg(src, idxg, idxs, idxgt, idxst, zrows, out,
            igA, isA, igB, isB, s0, s1, tg, ts, r0, r1, acc, g0, g1, zs, isem):
    cid = lax.axis_index("c")
    sid = lax.axis_index("s")
    wid = sid * NC + cid
    pltpu.async_copy(zrows, acc.at[pl.ds(sid * STRIPE, STRIPE)], zs)
    pltpu.sync_copy(idxg.at[wid, 0], igA)
    pltpu.sync_copy(idxs.at[wid, 0], isA)

    def stage(isv, c, dst):
        for k in range(B // 16):
            dst[pl.ds(16 * k, 16)] = isv[c, pl.ds(16 * k, 16)]

    stage(isA, 0, s0)
    pltpu.async_copy(src.at[igA.at[0]], r0, g0)
    pltpu.make_async_copy(zrows, acc.at[pl.ds(sid * STRIPE, STRIPE)], zs).wait()
    plsc.subcore_barrier()

    for p in range(PH):
        igv, isv = (igA, isA) if p % 2 == 0 else (igB, isB)
        ngv, nsv = (igB, isB) if p % 2 == 0 else (igA, isA)
        rX, sX, gX = (r0, s0, g0) if p % 2 == 0 else (r1, s1, g1)
        rY, sY, gY = (r1, s1, g1) if p % 2 == 0 else (r0, s0, g0)
        if p + 1 < PH:
            pltpu.async_copy(idxg.at[wid, p + 1], ngv, isem)
            pltpu.async_copy(idxs.at[wid, p + 1], nsv, isem)
        else:
            pltpu.async_copy(idxgt.at[wid], tg, isem)
            pltpu.async_copy(idxst.at[wid], ts, isem)

        def pair(g, carry2, igv=igv, isv=isv, rX=rX, sX=sX, gX=gX,
                 rY=rY, sY=sY, gY=gY):
            c0 = 2 * g
            stage(isv, c0 + 1, sY)
            pltpu.async_copy(src.at[igv.at[c0 + 1]], rY, gY)
            pltpu.make_async_copy(src.at[igv.at[c0]], rX, gX).wait()
            pltpu.sync_copy(rX, acc.at[sX], add=True)
            stage(isv, c0 + 2, sX)
            pltpu.async_copy(src.at[igv.at[c0 + 2]], rX, gX)
            pltpu.make_async_copy(src.at[igv.at[c0 + 1]], rY, gY).wait()
            pltpu.sync_copy(rY, acc.at[sY], add=True)
            return carry2

        lax.fori_loop(0, (PCH - 3) // 2, pair, 0)
        stage(isv, PCH - 2, sY)
        pltpu.async_copy(src.at[igv.at[PCH - 2]], rY, gY)
        pltpu.make_async_copy(src.at[igv.at[PCH - 3]], rX, gX).wait()
        pltpu.sync_copy(rX, acc.at[sX], add=True)
        stage(isv, PCH - 1, sX)
        pltpu.async_copy(src.at[igv.at[PCH - 1]], rX, gX)
        pltpu.make_async_copy(src.at[igv.at[PCH - 2]], rY, gY).wait()
        pltpu.sync_copy(rY, acc.at[sY], add=True)
        if p + 1 < PH:
            pltpu.make_async_copy(idxg.at[wid, p + 1], ngv, isem).wait()
            pltpu.make_async_copy(idxs.at[wid, p + 1], nsv, isem).wait()
            stage(nsv, 0, sY)
            pltpu.async_copy(src.at[ngv.at[0]], rY, gY)
        else:
            pltpu.make_async_copy(idxgt.at[wid], tg, isem).wait()
            pltpu.make_async_copy(idxst.at[wid], ts, isem).wait()
            pltpu.async_copy(src.at[tg], rY.at[pl.ds(0, TB)], gY)
        pltpu.make_async_copy(src.at[igv.at[PCH - 1]], rX, gX).wait()
        pltpu.sync_copy(rX, acc.at[sX], add=True)

    rT = r1 if (PH - 1) % 2 == 0 else r0
    gT = g1 if (PH - 1) % 2 == 0 else g0
    pltpu.make_async_copy(src.at[tg], rT.at[pl.ds(0, TB)], gT).wait()
    pltpu.sync_copy(rT.at[pl.ds(0, TB)], acc.at[ts], add=True)

    plsc.subcore_barrier()
    pltpu.sync_copy(acc.at[pl.ds(sid * STRIPE, STRIPE)],
                    out.at[cid, pl.ds(sid * STRIPE, STRIPE)])


@functools.partial(
    pl.kernel,
    out_type=(jax.ShapeDtypeStruct((NC, NP, 16), jnp.float32),
              jax.ShapeDtypeStruct((NC, NP, 16), jnp.float32)),
    mesh=_mesh,
    scratch_types=[
        pltpu.VMEM((PCH, B), jnp.int32),
        pltpu.VMEM((PCH, B), jnp.int32),
        pltpu.VMEM((PCH, B), jnp.int32),
        pltpu.VMEM((PCH, B), jnp.int32),
        pltpu.VMEM((B,), jnp.int32),
        pltpu.VMEM((B,), jnp.int32),
        pltpu.VMEM((B,), jnp.int32),
        pltpu.VMEM((B,), jnp.int32),
        pltpu.VMEM((TB,), jnp.int32),
        pltpu.VMEM((TB,), jnp.int32),
        pltpu.VMEM((B, 16), jnp.float32),
        pltpu.VMEM_SHARED((NP, 16), jnp.float32),
        pltpu.VMEM_SHARED((NP, 16), jnp.float32),
        pltpu.SemaphoreType.DMA,
        pltpu.SemaphoreType.DMA,
        pltpu.SemaphoreType.DMA,
        pltpu.SemaphoreType.DMA,
        pltpu.SemaphoreType.DMA,
    ],
)
def _sc_count(idxe, idxv, idxet, idxvt, z16, ones_hbm, oute, outv,
              ieA, ivA, ieB, ivB, se0, se1, sv0, sv1, te, tv, ones_v,
              acce, accv, ae0, av0, ae1, av1, isem):
    cid = lax.axis_index("c")
    sid = lax.axis_index("s")
    wid = sid * NC + cid
    pltpu.async_copy(z16, acce.at[pl.ds(sid * STRIPE, STRIPE)], ae0)
    pltpu.async_copy(z16, accv.at[pl.ds(sid * STRIPE, STRIPE)], av0)
    pltpu.sync_copy(ones_hbm, ones_v)
    pltpu.sync_copy(idxe.at[wid, 0], ieA)
    pltpu.sync_copy(idxv.at[wid, 0], ivA)
    pltpu.make_async_copy(z16, acce.at[pl.ds(sid * STRIPE, STRIPE)], ae0).wait()
    pltpu.make_async_copy(z16, accv.at[pl.ds(sid * STRIPE, STRIPE)], av0).wait()
    plsc.subcore_barrier()

    def stage(ref, c, dst):
        for k in range(B // 16):
            dst[pl.ds(16 * k, 16)] = ref[c, pl.ds(16 * k, 16)]

    def issue(se, sv, semee, semv):
        pltpu.async_copy(ones_v, acce.at[se], semee, add=True)
        pltpu.async_copy(ones_v, accv.at[sv], semv, add=True)

    def drain(se, sv, semee, semv):
        pltpu.make_async_copy(ones_v, acce.at[se], semee).wait()
        pltpu.make_async_copy(ones_v, accv.at[sv], semv).wait()

    for p in range(PH):
        iev, ivv = (ieA, ivA) if p % 2 == 0 else (ieB, ivB)
        nev, nvv = (ieB, ivB) if p % 2 == 0 else (ieA, ivA)
        stage(iev, 0, se0)
        stage(ivv, 0, sv0)
        issue(se0, sv0, ae0, av0)
        stage(iev, 1, se1)
        stage(ivv, 1, sv1)
        issue(se1, sv1, ae1, av1)
        if p + 1 < PH:
            pltpu.async_copy(idxe.at[wid, p + 1], nev, isem)
            pltpu.async_copy(idxv.at[wid, p + 1], nvv, isem)

        def pair(g, carry2, iev=iev, ivv=ivv):
            c = 2 * g + 2
            drain(se0, sv0, ae0, av0)
            stage(iev, c, se0)
            stage(ivv, c, sv0)
            issue(se0, sv0, ae0, av0)
            drain(se1, sv1, ae1, av1)
            stage(iev, c + 1, se1)
            stage(ivv, c + 1, sv1)
            issue(se1, sv1, ae1, av1)
            return carry2

        lax.fori_loop(0, (PCH - 3) // 2, pair, 0)
        drain(se0, sv0, ae0, av0)
        stage(iev, PCH - 1, se0)
        stage(ivv, PCH - 1, sv0)
        issue(se0, sv0, ae0, av0)
        drain(se0, sv0, ae0, av0)
        drain(se1, sv1, ae1, av1)
        if p + 1 < PH:
            pltpu.make_async_copy(idxe.at[wid, p + 1], nev, isem).wait()
            pltpu.make_async_copy(idxv.at[wid, p + 1], nvv, isem).wait()

    pltpu.sync_copy(idxet.at[wid], te)
    pltpu.sync_copy(idxvt.at[wid], tv)
    pltpu.sync_copy(ones_v.at[pl.ds(0, TB)], acce.at[te], add=True)
    pltpu.sync_copy(ones_v.at[pl.ds(0, TB)], accv.at[tv], add=True)

    plsc.subcore_barrier()
    pltpu.sync_copy(acce.at[pl.ds(sid * STRIPE, STRIPE)],
                    oute.at[cid, pl.ds(sid * STRIPE, STRIPE)])
    pltpu.sync_copy(accv.at[pl.ds(sid * STRIPE, STRIPE)],
                    outv.at[cid, pl.ds(sid * STRIPE, STRIPE)])


BR = 1000
GR = N // BR


def _linear_body(relu, x_ref, w_ref, b_ref, o_ref):
    acc = lax.dot_general(x_ref[...], w_ref[...], (((1,), (1,)), ((), ())),
                          preferred_element_type=jnp.float32)
    acc = acc + b_ref[...]
    o_ref[...] = jnp.maximum(acc, 0.0) if relu else acc


def _tc_linear(x, w, b, relu):
    return pl.pallas_call(
        functools.partial(_linear_body, relu),
        grid=(GR,),
        in_specs=[
            pl.BlockSpec((BR, x.shape[1]), lambda i: (i, 0)),
            pl.BlockSpec(w.shape, lambda i: (0, 0)),
            pl.BlockSpec((1, w.shape[0]), lambda i: (0, 0)),
        ],
        out_specs=pl.BlockSpec((BR, w.shape[0]), lambda i: (i, 0)),
        out_shape=jax.ShapeDtypeStruct((x.shape[0], w.shape[0]), jnp.float32),
    )(x, w, b.reshape(1, -1))


def _combine_body(p_ref, c_ref, o_ref):
    s = p_ref[0] + p_ref[1]
    cnt = c_ref[0, :, 0:1] + c_ref[1, :, 0:1]
    o_ref[...] = s * (1.0 / jnp.maximum(cnt, 1.0))


def _tc_combine(partials, cnts):
    return pl.pallas_call(
        _combine_body,
        grid=(GR,),
        in_specs=[
            pl.BlockSpec((NC, BR, H), lambda i: (0, i, 0)),
            pl.BlockSpec((NC, BR, 16), lambda i: (0, i, 0)),
        ],
        out_specs=pl.BlockSpec((BR, H), lambda i: (i, 0)),
        out_shape=jax.ShapeDtypeStruct((NE, H), jnp.float32),
    )(partials, cnts)


def _layer_body(beta, p_ref, c_ref, h0_ref, w_ref, o_ref):
    s = p_ref[0] + p_ref[1]
    cnt = c_ref[0, :, 0:1] + c_ref[1, :, 0:1]
    xv = s * (1.0 / jnp.maximum(cnt, 1.0))
    rn = jnp.sqrt(jnp.sum(xv * xv, axis=1, keepdims=True))
    xn = xv * jnp.where(rn > 0, 1.0 / rn, 0.0)
    xi = (1.0 - ALPHA) * xn + ALPHA * h0_ref[...]
    xw = lax.dot_general(xi, w_ref[...], (((1,), (1,)), ((), ())),
                         preferred_element_type=jnp.float32)
    o_ref[...] = jnp.maximum((1.0 - beta) * xi + beta * xw, 0.0)


def _final_body(beta, p_ref, c_ref, h0_ref, w_ref, wo_ref, bo_ref, o_ref):
    s = p_ref[0] + p_ref[1]
    cnt = c_ref[0, :, 0:1] + c_ref[1, :, 0:1]
    xv = s * (1.0 / jnp.maximum(cnt, 1.0))
    rn = jnp.sqrt(jnp.sum(xv * xv, axis=1, keepdims=True))
    xn = xv * jnp.where(rn > 0, 1.0 / rn, 0.0)
    xi = (1.0 - ALPHA) * xn + ALPHA * h0_ref[...]
    xw = lax.dot_general(xi, w_ref[...], (((1,), (1,)), ((), ())),
                         preferred_element_type=jnp.float32)
    h = jnp.maximum((1.0 - beta) * xi + beta * xw, 0.0)
    o_ref[...] = lax.dot_general(h, wo_ref[...], (((1,), (1,)), ((), ())),
                                 preferred_element_type=jnp.float32) + bo_ref[...]


def _tc_layer(partials, cnts, h0, w, beta):
    return pl.pallas_call(
        functools.partial(_layer_body, beta),
        grid=(GR,),
        in_specs=[
            pl.BlockSpec((NC, BR, H), lambda i: (0, i, 0)),
            pl.BlockSpec((NC, BR, 16), lambda i: (0, i, 0)),
            pl.BlockSpec((BR, H), lambda i: (i, 0)),
            pl.BlockSpec((H, H), lambda i: (0, 0)),
        ],
        out_specs=pl.BlockSpec((BR, H), lambda i: (i, 0)),
        out_shape=jax.ShapeDtypeStruct((N, H), jnp.float32),
    )(partials, cnts, h0, w)


def _tc_final(partials, cnts, h0, w, beta, wout, bout):
    return pl.pallas_call(
        functools.partial(_final_body, beta),
        grid=(GR,),
        in_specs=[
            pl.BlockSpec((NC, BR, H), lambda i: (0, i, 0)),
            pl.BlockSpec((NC, BR, 16), lambda i: (0, i, 0)),
            pl.BlockSpec((BR, H), lambda i: (i, 0)),
            pl.BlockSpec((H, H), lambda i: (0, 0)),
            pl.BlockSpec((C, H), lambda i: (0, 0)),
            pl.BlockSpec((1, C), lambda i: (0, 0)),
        ],
        out_specs=pl.BlockSpec((BR, C), lambda i: (i, 0)),
        out_shape=jax.ShapeDtypeStruct((N, C), jnp.float32),
    )(partials, cnts, h0, w, wout, bout.reshape(1, -1))


def kernel(x, edge_index, W0, b0, Ws, Wout, bout):
    vw = edge_index[0].reshape(NW, EPW)
    ew = edge_index[1].reshape(NW, EPW)
    vertex_m = vw[:, :FCH * B].reshape(NW, PH, PCH, B)
    edges_m = ew[:, :FCH * B].reshape(NW, PH, PCH, B)
    vertex_t = vw[:, FCH * B:]
    edges_t = ew[:, FCH * B:]
    zrows = jnp.zeros((STRIPE, H), jnp.float32)
    z16 = jnp.zeros((STRIPE, 16), jnp.float32)
    ones = jnp.ones((B, 16), jnp.float32)

    h = _tc_linear(x, W0, b0, relu=True)
    h0 = h
    ce, cv = _sc_count(edges_m, vertex_m, edges_t, vertex_t, z16, ones)
    for i in range(DEPTH):
        beta = math.log(0.5 / (i + 1) + 1.0)
        sep = _sc_agg(h, vertex_m, edges_m, vertex_t, edges_t, zrows)
        xe = _tc_combine(sep, ce)
        svp = _sc_agg(xe, edges_m, vertex_m, edges_t, vertex_t, zrows)
        if i < DEPTH - 1:
            h = _tc_layer(svp, cv, h0, Ws[i], beta)
        else:
            return _tc_final(svp, cv, h0, Ws[i], beta, Wout, bout)


# speedup vs baseline: 1.4674x; 1.0229x over previous
import functools
import math

import jax
import jax.numpy as jnp
from jax import lax
from jax.experimental import pallas as pl
from jax.experimental.pallas import tpu as pltpu
from jax.experimental.pallas import tpu_sc as plsc

N = 10000
E = 320000
NE = 10000
F = 128
H = 128
C = 128
DEPTH = 4
ALPHA = 0.1

NC = 2
NS = 16
NW = NC * NS
EPW = E // NW
B = 128
PH = 6
PCH = 13
FCH = PH * PCH
TB = EPW - FCH * B
NP = 10240
STRIPE = NP // NS

_mesh = plsc.VectorSubcoreMesh(core_axis_name="c", subcore_axis_name="s")


@functools.partial(
    pl.kernel,
    out_type=jax.ShapeDtypeStruct((NC, NP, H), jnp.float32),
    mesh=_mesh,
    scratch_types=[
        pltpu.VMEM((PCH, B), jnp.int32),
        pltpu.VMEM((PCH, B), jnp.int32),
        pltpu.VMEM((PCH, B), jnp.int32),
        pltpu.VMEM((PCH, B), jnp.int32),
        pltpu.VMEM((B,), jnp.int32),
        pltpu.VMEM((B,), jnp.int32),
        pltpu.VMEM((TB,), jnp.int32),
        pltpu.VMEM((TB,), jnp.int32),
        pltpu.VMEM((B, H), jnp.float32),
        pltpu.VMEM((B, H), jnp.float32),
        pltpu.VMEM_SHARED((NP, H), jnp.float32),
        pltpu.SemaphoreType.DMA,
        pltpu.SemaphoreType.DMA,
        pltpu.SemaphoreType.DMA,
        pltpu.SemaphoreType.DMA,
    ],
)
def _sc_agg(src, idxg, idxs, idxgt, idxst, zrows, out,
            igA, isA, igB, isB, s0, s1, tg, ts, r0, r1, acc, g0, g1, zs, isem):
    cid = lax.axis_index("c")
    sid = lax.axis_index("s")
    wid = sid * NC + cid
    pltpu.async_copy(zrows, acc.at[pl.ds(sid * STRIPE, STRIPE)], zs)
    pltpu.sync_copy(idxg.at[wid, 0], igA)
    pltpu.sync_copy(idxs.at[wid, 0], isA)

    def stage(isv, c, dst):
        for k in range(B // 16):
            dst[pl.ds(16 * k, 16)] = isv[c, pl.ds(16 * k, 16)]

    stage(isA, 0, s0)
    pltpu.async_copy(src.at[igA.at[0]], r0, g0)
    pltpu.make_async_copy(zrows, acc.at[pl.ds(sid * STRIPE, STRIPE)], zs).wait()
    plsc.subcore_barrier()

    for p in range(PH):
        igv, isv = (igA, isA) if p % 2 == 0 else (igB, isB)
        ngv, nsv = (igB, isB) if p % 2 == 0 else (igA, isA)
        rX, sX, gX = (r0, s0, g0) if p % 2 == 0 else (r1, s1, g1)
        rY, sY, gY = (r1, s1, g1) if p % 2 == 0 else (r0, s0, g0)
        if p + 1 < PH:
            pltpu.async_copy(idxg.at[wid, p + 1], ngv, isem)
            pltpu.async_copy(idxs.at[wid, p + 1], nsv, isem)
        else:
            pltpu.async_copy(idxgt.at[wid], tg, isem)
            pltpu.async_copy(idxst.at[wid], ts, isem)

        def pair(g, carry2, igv=igv, isv=isv, rX=rX, sX=sX, gX=gX,
                 rY=rY, sY=sY, gY=gY):
            c0 = 2 * g
            stage(isv, c0 + 1, sY)
            pltpu.async_copy(src.at[igv.at[c0 + 1]], rY, gY)
            pltpu.make_async_copy(src.at[igv.at[c0]], rX, gX).wait()
            pltpu.sync_copy(rX, acc.at[sX], add=True)
            stage(isv, c0 + 2, sX)
            pltpu.async_copy(src.at[igv.at[c0 + 2]], rX, gX)
            pltpu.make_async_copy(src.at[igv.at[c0 + 1]], rY, gY).wait()
            pltpu.sync_copy(rY, acc.at[sY], add=True)
            return carry2

        lax.fori_loop(0, (PCH - 3) // 2, pair, 0)
        stage(isv, PCH - 2, sY)
        pltpu.async_copy(src.at[igv.at[PCH - 2]], rY, gY)
        pltpu.make_async_copy(src.at[igv.at[PCH - 3]], rX, gX).wait()
        pltpu.sync_copy(rX, acc.at[sX], add=True)
        stage(isv, PCH - 1, sX)
        pltpu.async_copy(src.at[igv.at[PCH - 1]], rX, gX)
        pltpu.make_async_copy(src.at[igv.at[PCH - 2]], rY, gY).wait()
        pltpu.sync_copy(rY, acc.at[sY], add=True)
        if p + 1 < PH:
            pltpu.make_async_copy(idxg.at[wid, p + 1], ngv, isem).wait()
            pltpu.make_async_copy(idxs.at[wid, p + 1], nsv, isem).wait()
            stage(nsv, 0, sY)
            pltpu.async_copy(src.at[ngv.at[0]], rY, gY)
        else:
            pltpu.make_async_copy(idxgt.at[wid], tg, isem).wait()
            pltpu.make_async_copy(idxst.at[wid], ts, isem).wait()
            pltpu.async_copy(src.at[tg], rY.at[pl.ds(0, TB)], gY)
        pltpu.make_async_copy(src.at[igv.at[PCH - 1]], rX, gX).wait()
        pltpu.sync_copy(rX, acc.at[sX], add=True)

    rT = r1 if (PH - 1) % 2 == 0 else r0
    gT = g1 if (PH - 1) % 2 == 0 else g0
    pltpu.make_async_copy(src.at[tg], rT.at[pl.ds(0, TB)], gT).wait()
    pltpu.sync_copy(rT.at[pl.ds(0, TB)], acc.at[ts], add=True)

    plsc.subcore_barrier()
    pltpu.sync_copy(acc.at[pl.ds(sid * STRIPE, STRIPE)],
                    out.at[cid, pl.ds(sid * STRIPE, STRIPE)])


@functools.partial(
    pl.kernel,
    out_type=(jax.ShapeDtypeStruct((NC, NP, 16), jnp.float32),
              jax.ShapeDtypeStruct((NC, NP, 16), jnp.float32)),
    mesh=_mesh,
    scratch_types=[
        pltpu.VMEM((PCH, B), jnp.int32),
        pltpu.VMEM((PCH, B), jnp.int32),
        pltpu.VMEM((PCH, B), jnp.int32),
        pltpu.VMEM((PCH, B), jnp.int32),
        pltpu.VMEM((B,), jnp.int32),
        pltpu.VMEM((B,), jnp.int32),
        pltpu.VMEM((B,), jnp.int32),
        pltpu.VMEM((B,), jnp.int32),
        pltpu.VMEM((TB,), jnp.int32),
        pltpu.VMEM((TB,), jnp.int32),
        pltpu.VMEM((B, 16), jnp.float32),
        pltpu.VMEM_SHARED((NP, 16), jnp.float32),
        pltpu.VMEM_SHARED((NP, 16), jnp.float32),
        pltpu.SemaphoreType.DMA,
        pltpu.SemaphoreType.DMA,
        pltpu.SemaphoreType.DMA,
        pltpu.SemaphoreType.DMA,
        pltpu.SemaphoreType.DMA,
    ],
)
def _sc_count(idxe, idxv, idxet, idxvt, z16, ones_hbm, oute, outv,
              ieA, ivA, ieB, ivB, se0, se1, sv0, sv1, te, tv, ones_v,
              acce, accv, ae0, av0, ae1, av1, isem):
    cid = lax.axis_index("c")
    sid = lax.axis_index("s")
    wid = sid * NC + cid
    pltpu.async_copy(z16, acce.at[pl.ds(sid * STRIPE, STRIPE)], ae0)
    pltpu.async_copy(z16, accv.at[pl.ds(sid * STRIPE, STRIPE)], av0)
    pltpu.sync_copy(ones_hbm, ones_v)
    pltpu.sync_copy(idxe.at[wid, 0], ieA)
    pltpu.sync_copy(idxv.at[wid, 0], ivA)
    pltpu.make_async_copy(z16, acce.at[pl.ds(sid * STRIPE, STRIPE)], ae0).wait()
    pltpu.make_async_copy(z16, accv.at[pl.ds(sid * STRIPE, STRIPE)], av0).wait()
    plsc.subcore_barrier()

    def stage(ref, c, dst):
        for k in range(B // 16):
            dst[pl.ds(16 * k, 16)] = ref[c, pl.ds(16 * k, 16)]

    def issue(se, sv, semee, semv):
        pltpu.async_copy(ones_v, acce.at[se], semee, add=True)
        pltpu.async_copy(ones_v, accv.at[sv], semv, add=True)

    def drain(se, sv, semee, semv):
        pltpu.make_async_copy(ones_v, acce.at[se], semee).wait()
        pltpu.make_async_copy(ones_v, accv.at[sv], semv).wait()

    for p in range(PH):
        iev, ivv = (ieA, ivA) if p % 2 == 0 else (ieB, ivB)
        nev, nvv = (ieB, ivB) if p % 2 == 0 else (ieA, ivA)
        stage(iev, 0, se0)
        stage(ivv, 0, sv0)
        issue(se0, sv0, ae0, av0)
        stage(iev, 1, se1)
        stage(ivv, 1, sv1)
        issue(se1, sv1, ae1, av1)
        if p + 1 < PH:
            pltpu.async_copy(idxe.at[wid, p + 1], nev, isem)
            pltpu.async_copy(idxv.at[wid, p + 1], nvv, isem)

        def pair(g, carry2, iev=iev, ivv=ivv):
            c = 2 * g + 2
            drain(se0, sv0, ae0, av0)
            stage(iev, c, se0)
            stage(ivv, c, sv0)
            issue(se0, sv0, ae0, av0)
            drain(se1, sv1, ae1, av1)
            stage(iev, c + 1, se1)
            stage(ivv, c + 1, sv1)
            issue(se1, sv1, ae1, av1)
            return carry2

        lax.fori_loop(0, (PCH - 3) // 2, pair, 0)
        drain(se0, sv0, ae0, av0)
        stage(iev, PCH - 1, se0)
        stage(ivv, PCH - 1, sv0)
        issue(se0, sv0, ae0, av0)
        drain(se0, sv0, ae0, av0)
        drain(se1, sv1, ae1, av1)
        if p + 1 < PH:
            pltpu.make_async_copy(idxe.at[wid, p + 1], nev, isem).wait()
            pltpu.make_async_copy(idxv.at[wid, p + 1], nvv, isem).wait()

    pltpu.sync_copy(idxet.at[wid], te)
    pltpu.sync_copy(idxvt.at[wid], tv)
    pltpu.sync_copy(ones_v.at[pl.ds(0, TB)], acce.at[te], add=True)
    pltpu.sync_copy(ones_v.at[pl.ds(0, TB)], accv.at[tv], add=True)

    plsc.subcore_barrier()
    pltpu.sync_copy(acce.at[pl.ds(sid * STRIPE, STRIPE)],
                    oute.at[cid, pl.ds(sid * STRIPE, STRIPE)])
    pltpu.sync_copy(accv.at[pl.ds(sid * STRIPE, STRIPE)],
                    outv.at[cid, pl.ds(sid * STRIPE, STRIPE)])


BR = 2000
GR = N // BR


def _linear_body(relu, x_ref, w_ref, b_ref, o_ref):
    acc = lax.dot_general(x_ref[...], w_ref[...], (((1,), (1,)), ((), ())),
                          preferred_element_type=jnp.float32)
    acc = acc + b_ref[...]
    o_ref[...] = jnp.maximum(acc, 0.0) if relu else acc


def _tc_linear(x, w, b, relu):
    return pl.pallas_call(
        functools.partial(_linear_body, relu),
        grid=(GR,),
        in_specs=[
            pl.BlockSpec((BR, x.shape[1]), lambda i: (i, 0)),
            pl.BlockSpec(w.shape, lambda i: (0, 0)),
            pl.BlockSpec((1, w.shape[0]), lambda i: (0, 0)),
        ],
        out_specs=pl.BlockSpec((BR, w.shape[0]), lambda i: (i, 0)),
        out_shape=jax.ShapeDtypeStruct((x.shape[0], w.shape[0]), jnp.float32),
    )(x, w, b.reshape(1, -1))


def _combine_body(p_ref, c_ref, o_ref):
    s = p_ref[0] + p_ref[1]
    cnt = c_ref[0, :, 0:1] + c_ref[1, :, 0:1]
    o_ref[...] = s * (1.0 / jnp.maximum(cnt, 1.0))


def _tc_combine(partials, cnts):
    return pl.pallas_call(
        _combine_body,
        grid=(GR,),
        in_specs=[
            pl.BlockSpec((NC, BR, H), lambda i: (0, i, 0)),
            pl.BlockSpec((NC, BR, 16), lambda i: (0, i, 0)),
        ],
        out_specs=pl.BlockSpec((BR, H), lambda i: (i, 0)),
        out_shape=jax.ShapeDtypeStruct((NE, H), jnp.float32),
    )(partials, cnts)


def _layer_body(beta, p_ref, c_ref, h0_ref, w_ref, o_ref):
    s = p_ref[0] + p_ref[1]
    cnt = c_ref[0, :, 0:1] + c_ref[1, :, 0:1]
    xv = s * (1.0 / jnp.maximum(cnt, 1.0))
    rn = jnp.sqrt(jnp.sum(xv * xv, axis=1, keepdims=True))
    xn = xv * jnp.where(rn > 0, 1.0 / rn, 0.0)
    xi = (1.0 - ALPHA) * xn + ALPHA * h0_ref[...]
    xw = lax.dot_general(xi, w_ref[...], (((1,), (1,)), ((), ())),
                         preferred_element_type=jnp.float32)
    o_ref[...] = jnp.maximum((1.0 - beta) * xi + beta * xw, 0.0)


def _final_body(beta, p_ref, c_ref, h0_ref, w_ref, wo_ref, bo_ref, o_ref):
    s = p_ref[0] + p_ref[1]
    cnt = c_ref[0, :, 0:1] + c_ref[1, :, 0:1]
    xv = s * (1.0 / jnp.maximum(cnt, 1.0))
    rn = jnp.sqrt(jnp.sum(xv * xv, axis=1, keepdims=True))
    xn = xv * jnp.where(rn > 0, 1.0 / rn, 0.0)
    xi = (1.0 - ALPHA) * xn + ALPHA * h0_ref[...]
    xw = lax.dot_general(xi, w_ref[...], (((1,), (1,)), ((), ())),
                         preferred_element_type=jnp.float32)
    h = jnp.maximum((1.0 - beta) * xi + beta * xw, 0.0)
    o_ref[...] = lax.dot_general(h, wo_ref[...], (((1,), (1,)), ((), ())),
                                 preferred_element_type=jnp.float32) + bo_ref[...]


def _tc_layer(partials, cnts, h0, w, beta):
    return pl.pallas_call(
        functools.partial(_layer_body, beta),
        grid=(GR,),
        in_specs=[
            pl.BlockSpec((NC, BR, H), lambda i: (0, i, 0)),
            pl.BlockSpec((NC, BR, 16), lambda i: (0, i, 0)),
            pl.BlockSpec((BR, H), lambda i: (i, 0)),
            pl.BlockSpec((H, H), lambda i: (0, 0)),
        ],
        out_specs=pl.BlockSpec((BR, H), lambda i: (i, 0)),
        out_shape=jax.ShapeDtypeStruct((N, H), jnp.float32),
    )(partials, cnts, h0, w)


def _tc_final(partials, cnts, h0, w, beta, wout, bout):
    return pl.pallas_call(
        functools.partial(_final_body, beta),
        grid=(GR,),
        in_specs=[
            pl.BlockSpec((NC, BR, H), lambda i: (0, i, 0)),
            pl.BlockSpec((NC, BR, 16), lambda i: (0, i, 0)),
            pl.BlockSpec((BR, H), lambda i: (i, 0)),
            pl.BlockSpec((H, H), lambda i: (0, 0)),
            pl.BlockSpec((C, H), lambda i: (0, 0)),
            pl.BlockSpec((1, C), lambda i: (0, 0)),
        ],
        out_specs=pl.BlockSpec((BR, C), lambda i: (i, 0)),
        out_shape=jax.ShapeDtypeStruct((N, C), jnp.float32),
    )(partials, cnts, h0, w, wout, bout.reshape(1, -1))


def kernel(x, edge_index, W0, b0, Ws, Wout, bout):
    vw = edge_index[0].reshape(NW, EPW)
    ew = edge_index[1].reshape(NW, EPW)
    vertex_m = vw[:, :FCH * B].reshape(NW, PH, PCH, B)
    edges_m = ew[:, :FCH * B].reshape(NW, PH, PCH, B)
    vertex_t = vw[:, FCH * B:]
    edges_t = ew[:, FCH * B:]
    zrows = jnp.zeros((STRIPE, H), jnp.float32)
    z16 = jnp.zeros((STRIPE, 16), jnp.float32)
    ones = jnp.ones((B, 16), jnp.float32)

    h = _tc_linear(x, W0, b0, relu=True)
    h0 = h
    ce, cv = _sc_count(edges_m, vertex_m, edges_t, vertex_t, z16, ones)
    for i in range(DEPTH):
        beta = math.log(0.5 / (i + 1) + 1.0)
        sep = _sc_agg(h, vertex_m, edges_m, vertex_t, edges_t, zrows)
        xe = _tc_combine(sep, ce)
        svp = _sc_agg(xe, edges_m, vertex_m, edges_t, vertex_t, zrows)
        if i < DEPTH - 1:
            h = _tc_layer(svp, cv, h0, Ws[i], beta)
        else:
            return _tc_final(svp, cv, h0, Ws[i], beta, Wout, bout)


# speedup vs baseline: 1.4748x; 1.0050x over previous
import functools
import math

import jax
import jax.numpy as jnp
from jax import lax
from jax.experimental import pallas as pl
from jax.experimental.pallas import tpu as pltpu
from jax.experimental.pallas import tpu_sc as plsc

N = 10000
E = 320000
NE = 10000
F = 128
H = 128
C = 128
DEPTH = 4
ALPHA = 0.1

NC = 2
NS = 16
NW = NC * NS
EPW = E // NW
B = 128
PH = 6
PCH = 13
FCH = PH * PCH
TB = EPW - FCH * B
NP = 10240
STRIPE = NP // NS

_mesh = plsc.VectorSubcoreMesh(core_axis_name="c", subcore_axis_name="s")


@functools.partial(
    pl.kernel,
    out_type=jax.ShapeDtypeStruct((NC, NP, H), jnp.float32),
    mesh=_mesh,
    scratch_types=[
        pltpu.VMEM((PCH, B), jnp.int32),
        pltpu.VMEM((PCH, B), jnp.int32),
        pltpu.VMEM((PCH, B), jnp.int32),
        pltpu.VMEM((PCH, B), jnp.int32),
        pltpu.VMEM((B,), jnp.int32),
        pltpu.VMEM((B,), jnp.int32),
        pltpu.VMEM((TB,), jnp.int32),
        pltpu.VMEM((TB,), jnp.int32),
        pltpu.VMEM((B, H), jnp.float32),
        pltpu.VMEM((B, H), jnp.float32),
        pltpu.VMEM_SHARED((NP, H), jnp.float32),
        pltpu.SemaphoreType.DMA,
        pltpu.SemaphoreType.DMA,
        pltpu.SemaphoreType.DMA,
        pltpu.SemaphoreType.DMA,
    ],
)
def _sc_agg(src, idxg, idxs, idxgt, idxst, zrows, out,
            igA, isA, igB, isB, s0, s1, tg, ts, r0, r1, acc, g0, g1, zs, isem):
    cid = lax.axis_index("c")
    sid = lax.axis_index("s")
    wid = sid * NC + cid
    pltpu.async_copy(zrows, acc.at[pl.ds(sid * STRIPE, STRIPE)], zs)
    pltpu.sync_copy(idxg.at[wid, 0], igA)
    pltpu.sync_copy(idxs.at[wid, 0], isA)

    def stage(isv, c, dst):
        for k in range(B // 16):
            dst[pl.ds(16 * k, 16)] = isv[c, pl.ds(16 * k, 16)]

    stage(isA, 0, s0)
    pltpu.async_copy(src.at[igA.at[0]], r0, g0)
    pltpu.make_async_copy(zrows, acc.at[pl.ds(sid * STRIPE, STRIPE)], zs).wait()
    plsc.subcore_barrier()

    for p in range(PH):
        igv, isv = (igA, isA) if p % 2 == 0 else (igB, isB)
        ngv, nsv = (igB, isB) if p % 2 == 0 else (igA, isA)
        rX, sX, gX = (r0, s0, g0) if p % 2 == 0 else (r1, s1, g1)
        rY, sY, gY = (r1, s1, g1) if p % 2 == 0 else (r0, s0, g0)
        if p + 1 < PH:
            pltpu.async_copy(idxg.at[wid, p + 1], ngv, isem)
            pltpu.async_copy(idxs.at[wid, p + 1], nsv, isem)
        else:
            pltpu.async_copy(idxgt.at[wid], tg, isem)
            pltpu.async_copy(idxst.at[wid], ts, isem)

        def pair(g, carry2, igv=igv, isv=isv, rX=rX, sX=sX, gX=gX,
                 rY=rY, sY=sY, gY=gY):
            c0 = 2 * g
            stage(isv, c0 + 1, sY)
            pltpu.async_copy(src.at[igv.at[c0 + 1]], rY, gY)
            pltpu.make_async_copy(src.at[igv.at[c0]], rX, gX).wait()
            pltpu.sync_copy(rX, acc.at[sX], add=True)
            stage(isv, c0 + 2, sX)
            pltpu.async_copy(src.at[igv.at[c0 + 2]], rX, gX)
            pltpu.make_async_copy(src.at[igv.at[c0 + 1]], rY, gY).wait()
            pltpu.sync_copy(rY, acc.at[sY], add=True)
            return carry2

        lax.fori_loop(0, (PCH - 3) // 2, pair, 0)
        stage(isv, PCH - 2, sY)
        pltpu.async_copy(src.at[igv.at[PCH - 2]], rY, gY)
        pltpu.make_async_copy(src.at[igv.at[PCH - 3]], rX, gX).wait()
        pltpu.sync_copy(rX, acc.at[sX], add=True)
        stage(isv, PCH - 1, sX)
        pltpu.async_copy(src.at[igv.at[PCH - 1]], rX, gX)
        pltpu.make_async_copy(src.at[igv.at[PCH - 2]], rY, gY).wait()
        pltpu.sync_copy(rY, acc.at[sY], add=True)
        if p + 1 < PH:
            pltpu.make_async_copy(idxg.at[wid, p + 1], ngv, isem).wait()
            pltpu.make_async_copy(idxs.at[wid, p + 1], nsv, isem).wait()
            stage(nsv, 0, sY)
            pltpu.async_copy(src.at[ngv.at[0]], rY, gY)
        else:
            pltpu.make_async_copy(idxgt.at[wid], tg, isem).wait()
            pltpu.make_async_copy(idxst.at[wid], ts, isem).wait()
            pltpu.async_copy(src.at[tg], rY.at[pl.ds(0, TB)], gY)
        pltpu.make_async_copy(src.at[igv.at[PCH - 1]], rX, gX).wait()
        pltpu.sync_copy(rX, acc.at[sX], add=True)

    rT = r1 if (PH - 1) % 2 == 0 else r0
    gT = g1 if (PH - 1) % 2 == 0 else g0
    pltpu.make_async_copy(src.at[tg], rT.at[pl.ds(0, TB)], gT).wait()
    pltpu.sync_copy(rT.at[pl.ds(0, TB)], acc.at[ts], add=True)

    plsc.subcore_barrier()
    pltpu.sync_copy(acc.at[pl.ds(sid * STRIPE, STRIPE)],
                    out.at[cid, pl.ds(sid * STRIPE, STRIPE)])


@functools.partial(
    pl.kernel,
    out_type=(jax.ShapeDtypeStruct((NC, NP, 16), jnp.float32),
              jax.ShapeDtypeStruct((NC, NP, 16), jnp.float32)),
    mesh=_mesh,
    scratch_types=[
        pltpu.VMEM((PCH, B), jnp.int32),
        pltpu.VMEM((PCH, B), jnp.int32),
        pltpu.VMEM((PCH, B), jnp.int32),
        pltpu.VMEM((PCH, B), jnp.int32),
        pltpu.VMEM((B,), jnp.int32),
        pltpu.VMEM((B,), jnp.int32),
        pltpu.VMEM((B,), jnp.int32),
        pltpu.VMEM((B,), jnp.int32),
        pltpu.VMEM((TB,), jnp.int32),
        pltpu.VMEM((TB,), jnp.int32),
        pltpu.VMEM((B, 16), jnp.float32),
        pltpu.VMEM_SHARED((NP, 16), jnp.float32),
        pltpu.VMEM_SHARED((NP, 16), jnp.float32),
        pltpu.SemaphoreType.DMA,
        pltpu.SemaphoreType.DMA,
        pltpu.SemaphoreType.DMA,
        pltpu.SemaphoreType.DMA,
        pltpu.SemaphoreType.DMA,
    ],
)
def _sc_count(idxe, idxv, idxet, idxvt, z16, ones_hbm, oute, outv,
              ieA, ivA, ieB, ivB, se0, se1, sv0, sv1, te, tv, ones_v,
              acce, accv, ae0, av0, ae1, av1, isem):
    cid = lax.axis_index("c")
    sid = lax.axis_index("s")
    wid = sid * NC + cid
    pltpu.async_copy(z16, acce.at[pl.ds(sid * STRIPE, STRIPE)], ae0)
    pltpu.async_copy(z16, accv.at[pl.ds(sid * STRIPE, STRIPE)], av0)
    pltpu.sync_copy(ones_hbm, ones_v)
    pltpu.sync_copy(idxe.at[wid, 0], ieA)
    pltpu.sync_copy(idxv.at[wid, 0], ivA)
    pltpu.make_async_copy(z16, acce.at[pl.ds(sid * STRIPE, STRIPE)], ae0).wait()
    pltpu.make_async_copy(z16, accv.at[pl.ds(sid * STRIPE, STRIPE)], av0).wait()
    plsc.subcore_barrier()

    def stage(ref, c, dst):
        for k in range(B // 16):
            dst[pl.ds(16 * k, 16)] = ref[c, pl.ds(16 * k, 16)]

    def issue(se, sv, semee, semv):
        pltpu.async_copy(ones_v, acce.at[se], semee, add=True)
        pltpu.async_copy(ones_v, accv.at[sv], semv, add=True)

    def drain(se, sv, semee, semv):
        pltpu.make_async_copy(ones_v, acce.at[se], semee).wait()
        pltpu.make_async_copy(ones_v, accv.at[sv], semv).wait()

    for p in range(PH):
        iev, ivv = (ieA, ivA) if p % 2 == 0 else (ieB, ivB)
        nev, nvv = (ieB, ivB) if p % 2 == 0 else (ieA, ivA)
        stage(iev, 0, se0)
        stage(ivv, 0, sv0)
        issue(se0, sv0, ae0, av0)
        stage(iev, 1, se1)
        stage(ivv, 1, sv1)
        issue(se1, sv1, ae1, av1)
        if p + 1 < PH:
            pltpu.async_copy(idxe.at[wid, p + 1], nev, isem)
            pltpu.async_copy(idxv.at[wid, p + 1], nvv, isem)

        def pair(g, carry2, iev=iev, ivv=ivv):
            c = 2 * g + 2
            drain(se0, sv0, ae0, av0)
            stage(iev, c, se0)
            stage(ivv, c, sv0)
            issue(se0, sv0, ae0, av0)
            drain(se1, sv1, ae1, av1)
            stage(iev, c + 1, se1)
            stage(ivv, c + 1, sv1)
            issue(se1, sv1, ae1, av1)
            return carry2

        lax.fori_loop(0, (PCH - 3) // 2, pair, 0)
        drain(se0, sv0, ae0, av0)
        stage(iev, PCH - 1, se0)
        stage(ivv, PCH - 1, sv0)
        issue(se0, sv0, ae0, av0)
        drain(se0, sv0, ae0, av0)
        drain(se1, sv1, ae1, av1)
        if p + 1 < PH:
            pltpu.make_async_copy(idxe.at[wid, p + 1], nev, isem).wait()
            pltpu.make_async_copy(idxv.at[wid, p + 1], nvv, isem).wait()

    pltpu.sync_copy(idxet.at[wid], te)
    pltpu.sync_copy(idxvt.at[wid], tv)
    pltpu.sync_copy(ones_v.at[pl.ds(0, TB)], acce.at[te], add=True)
    pltpu.sync_copy(ones_v.at[pl.ds(0, TB)], accv.at[tv], add=True)

    plsc.subcore_barrier()
    pltpu.sync_copy(acce.at[pl.ds(sid * STRIPE, STRIPE)],
                    oute.at[cid, pl.ds(sid * STRIPE, STRIPE)])
    pltpu.sync_copy(accv.at[pl.ds(sid * STRIPE, STRIPE)],
                    outv.at[cid, pl.ds(sid * STRIPE, STRIPE)])


BR = 5000
GR = N // BR


def _linear_body(relu, x_ref, w_ref, b_ref, o_ref):
    acc = lax.dot_general(x_ref[...], w_ref[...], (((1,), (1,)), ((), ())),
                          preferred_element_type=jnp.float32)
    acc = acc + b_ref[...]
    o_ref[...] = jnp.maximum(acc, 0.0) if relu else acc


def _tc_linear(x, w, b, relu):
    return pl.pallas_call(
        functools.partial(_linear_body, relu),
        grid=(GR,),
        in_specs=[
            pl.BlockSpec((BR, x.shape[1]), lambda i: (i, 0)),
            pl.BlockSpec(w.shape, lambda i: (0, 0)),
            pl.BlockSpec((1, w.shape[0]), lambda i: (0, 0)),
        ],
        out_specs=pl.BlockSpec((BR, w.shape[0]), lambda i: (i, 0)),
        out_shape=jax.ShapeDtypeStruct((x.shape[0], w.shape[0]), jnp.float32),
    )(x, w, b.reshape(1, -1))


def _combine_body(p_ref, c_ref, o_ref):
    s = p_ref[0] + p_ref[1]
    cnt = c_ref[0, :, 0:1] + c_ref[1, :, 0:1]
    o_ref[...] = s * (1.0 / jnp.maximum(cnt, 1.0))


def _tc_combine(partials, cnts):
    return pl.pallas_call(
        _combine_body,
        grid=(GR,),
        in_specs=[
            pl.BlockSpec((NC, BR, H), lambda i: (0, i, 0)),
            pl.BlockSpec((NC, BR, 16), lambda i: (0, i, 0)),
        ],
        out_specs=pl.BlockSpec((BR, H), lambda i: (i, 0)),
        out_shape=jax.ShapeDtypeStruct((NE, H), jnp.float32),
    )(partials, cnts)


def _layer_body(beta, p_ref, c_ref, h0_ref, w_ref, o_ref):
    s = p_ref[0] + p_ref[1]
    cnt = c_ref[0, :, 0:1] + c_ref[1, :, 0:1]
    xv = s * (1.0 / jnp.maximum(cnt, 1.0))
    rn = jnp.sqrt(jnp.sum(xv * xv, axis=1, keepdims=True))
    xn = xv * jnp.where(rn > 0, 1.0 / rn, 0.0)
    xi = (1.0 - ALPHA) * xn + ALPHA * h0_ref[...]
    xw = lax.dot_general(xi, w_ref[...], (((1,), (1,)), ((), ())),
                         preferred_element_type=jnp.float32)
    o_ref[...] = jnp.maximum((1.0 - beta) * xi + beta * xw, 0.0)


def _final_body(beta, p_ref, c_ref, h0_ref, w_ref, wo_ref, bo_ref, o_ref):
    s = p_ref[0] + p_ref[1]
    cnt = c_ref[0, :, 0:1] + c_ref[1, :, 0:1]
    xv = s * (1.0 / jnp.maximum(cnt, 1.0))
    rn = jnp.sqrt(jnp.sum(xv * xv, axis=1, keepdims=True))
    xn = xv * jnp.where(rn > 0, 1.0 / rn, 0.0)
    xi = (1.0 - ALPHA) * xn + ALPHA * h0_ref[...]
    xw = lax.dot_general(xi, w_ref[...], (((1,), (1,)), ((), ())),
                         preferred_element_type=jnp.float32)
    h = jnp.maximum((1.0 - beta) * xi + beta * xw, 0.0)
    o_ref[...] = lax.dot_general(h, wo_ref[...], (((1,), (1,)), ((), ())),
                                 preferred_element_type=jnp.float32) + bo_ref[...]


def _tc_layer(partials, cnts, h0, w, beta):
    return pl.pallas_call(
        functools.partial(_layer_body, beta),
        grid=(GR,),
        in_specs=[
            pl.BlockSpec((NC, BR, H), lambda i: (0, i, 0)),
            pl.BlockSpec((NC, BR, 16), lambda i: (0, i, 0)),
            pl.BlockSpec((BR, H), lambda i: (i, 0)),
            pl.BlockSpec((H, H), lambda i: (0, 0)),
        ],
        out_specs=pl.BlockSpec((BR, H), lambda i: (i, 0)),
        out_shape=jax.ShapeDtypeStruct((N, H), jnp.float32),
    )(partials, cnts, h0, w)


def _tc_final(partials, cnts, h0, w, beta, wout, bout):
    return pl.pallas_call(
        functools.partial(_final_body, beta),
        grid=(GR,),
        in_specs=[
            pl.BlockSpec((NC, BR, H), lambda i: (0, i, 0)),
            pl.BlockSpec((NC, BR, 16), lambda i: (0, i, 0)),
            pl.BlockSpec((BR, H), lambda i: (i, 0)),
            pl.BlockSpec((H, H), lambda i: (0, 0)),
            pl.BlockSpec((C, H), lambda i: (0, 0)),
            pl.BlockSpec((1, C), lambda i: (0, 0)),
        ],
        out_specs=pl.BlockSpec((BR, C), lambda i: (i, 0)),
        out_shape=jax.ShapeDtypeStruct((N, C), jnp.float32),
    )(partials, cnts, h0, w, wout, bout.reshape(1, -1))


def kernel(x, edge_index, W0, b0, Ws, Wout, bout):
    vw = edge_index[0].reshape(NW, EPW)
    ew = edge_index[1].reshape(NW, EPW)
    vertex_m = vw[:, :FCH * B].reshape(NW, PH, PCH, B)
    edges_m = ew[:, :FCH * B].reshape(NW, PH, PCH, B)
    vertex_t = vw[:, FCH * B:]
    edges_t = ew[:, FCH * B:]
    zrows = jnp.zeros((STRIPE, H), jnp.float32)
    z16 = jnp.zeros((STRIPE, 16), jnp.float32)
    ones = jnp.ones((B, 16), jnp.float32)

    h = _tc_linear(x, W0, b0, relu=True)
    h0 = h
    ce, cv = _sc_count(edges_m, vertex_m, edges_t, vertex_t, z16, ones)
    for i in range(DEPTH):
        beta = math.log(0.5 / (i + 1) + 1.0)
        sep = _sc_agg(h, vertex_m, edges_m, vertex_t, edges_t, zrows)
        xe = _tc_combine(sep, ce)
        svp = _sc_agg(xe, edges_m, vertex_m, edges_t, vertex_t, zrows)
        if i < DEPTH - 1:
            h = _tc_layer(svp, cv, h0, Ws[i], beta)
        else:
            return _tc_final(svp, cv, h0, Ws[i], beta, Wout, bout)
